# Initial kernel scaffold; baseline (speedup 1.0000x reference)
import jax
import jax.numpy as jnp
from jax.experimental import pallas as pl


def kernel(x_user, x_sub, edge_index_us, edge_index_su, W1_us, b1_us, W1_su, b1_su, W2_us, b2_us, W2_su, b2_su):
    raise NotImplementedError("write your pallas kernel here")


# speedup vs baseline: 3.9637x; 3.9637x over previous
import functools

import jax
import jax.numpy as jnp
from jax import lax
from jax.experimental import pallas as pl
from jax.experimental.pallas import tpu as pltpu
from jax.experimental.pallas import tpu_sc as plsc

NU = 10000
E = 160000
D_IN = 256
D_H = 256
D_OUT = 128

NTILES = 16
CHUNK = 128
NCHUNK = 80
EPT = NCHUNK * CHUNK
E_PAD = NTILES * EPT
NPAD = E_PAD - E
TBL_N = NU + NPAD
NTRASH = 240
NU_PAD = 10240
ACC_H = 14336
ZCH_H = ACC_H // NTILES
RPT = NU_PAD // NTILES
RSTG = 64
IDXB = 8

_f32 = jnp.float32
_i32 = jnp.int32


def _pad_src(idx):
    pad = jnp.arange(NU, NU + NPAD, dtype=_i32)
    return jnp.concatenate([idx.astype(_i32), pad]).reshape(NTILES, NCHUNK, CHUNK)


def _pad_dst(idx):
    pad = NU + jnp.arange(NPAD, dtype=_i32) % NTRASH
    return jnp.concatenate([idx.astype(_i32), pad]).reshape(NTILES, NCHUNK, CHUNK)


def _deg_body(sp, dp, out, acc_s, acc_d, idx_s, idx_d, ones_v, zbuf):
    c = lax.axis_index("c")
    t = lax.axis_index("s")

    def _zi(i, carry):
        zbuf[pl.ds(i * 16, 16)] = jnp.zeros((16,), _f32)
        return carry
    lax.fori_loop(0, ZCH_H // 16, _zi, 0)

    def _oi(i, carry):
        ones_v[pl.ds(i * 16, 16)] = jnp.ones((16,), _f32)
        return carry
    lax.fori_loop(0, CHUNK // 16, _oi, 0)

    zoff = pl.multiple_of(t * ZCH_H, ZCH_H)
    pltpu.sync_copy(zbuf, acc_s.at[pl.ds(zoff, ZCH_H)])
    pltpu.sync_copy(zbuf, acc_d.at[pl.ds(zoff, ZCH_H)])
    w = c * NTILES + t
    pltpu.sync_copy(sp.at[w], idx_s)
    pltpu.sync_copy(dp.at[w], idx_d)
    plsc.subcore_barrier()

    def _ch(j, carry):
        pltpu.sync_copy(ones_v, acc_s.at[idx_s.at[j]], add=True)
        pltpu.sync_copy(ones_v, acc_d.at[idx_d.at[j]], add=True)
        return carry
    lax.fori_loop(0, NCHUNK, _ch, 0)
    plsc.subcore_barrier()

    pltpu.sync_copy(acc_s.at[pl.ds(zoff, ZCH_H)], out.at[2 * c, 0, pl.ds(zoff, ZCH_H)])
    pltpu.sync_copy(acc_d.at[pl.ds(zoff, ZCH_H)], out.at[2 * c + 1, 0, pl.ds(zoff, ZCH_H)])


def _degrees(sp12, dp12):
    mesh = plsc.VectorSubcoreMesh(core_axis_name="c", subcore_axis_name="s")
    return pl.kernel(
        _deg_body,
        out_type=jax.ShapeDtypeStruct((4, 1, ACC_H), _f32),
        mesh=mesh,
        scratch_types=[
            pltpu.VMEM_SHARED((ACC_H,), _f32),
            pltpu.VMEM_SHARED((ACC_H,), _f32),
            pltpu.VMEM((NCHUNK, CHUNK), _i32),
            pltpu.VMEM((NCHUNK, CHUNK), _i32),
            pltpu.VMEM((CHUNK,), _f32),
            pltpu.VMEM((ZCH_H,), _f32),
        ],
    )(sp12, dp12)


def _msg_body(relu_ns, tbl, sp, dp, nd, ns, bias, out,
              acc, rows_v, idx_s, idx_d, zbuf, stage, ndv, nsv, bv, sem):
    c = lax.axis_index("c")
    t = lax.axis_index("s")
    w = c * NTILES + t

    def _zi(i, carry):
        for g in range(8):
            zbuf[i, pl.ds(g * 16, 16)] = jnp.zeros((16,), _f32)
        return carry
    lax.fori_loop(0, 16, _zi, 0)

    def _za(k, carry):
        off = pl.multiple_of(t * RPT + k * 16, 16)
        pltpu.sync_copy(zbuf, acc.at[pl.ds(off, 16)])
        return carry
    lax.fori_loop(0, RPT // 16, _za, 0)

    tstart = pl.multiple_of(t * RPT, RPT)
    pltpu.sync_copy(nd.at[c, 0, pl.ds(tstart, RPT)], ndv.at[pl.ds(0, RPT)])
    if relu_ns:
        pltpu.sync_copy(ns.at[c, 0, pl.ds(tstart, RPT)], nsv.at[pl.ds(0, RPT)])
    pltpu.sync_copy(bias.at[c, 0], bv)
    plsc.subcore_barrier()

    def _win(jw, carry):
        woff = pl.multiple_of(jw * IDXB, IDXB)
        pltpu.sync_copy(sp.at[w, pl.ds(woff, IDXB)], idx_s)
        pltpu.sync_copy(dp.at[w, pl.ds(woff, IDXB)], idx_d)

        def _ch(k, carry2):
            pltpu.async_copy(tbl.at[c].at[idx_s.at[k]], rows_v, sem).wait()
            pltpu.sync_copy(rows_v, acc.at[idx_d.at[k]], add=True)
            return carry2
        lax.fori_loop(0, IDXB, _ch, 0)
        return carry
    lax.fori_loop(0, NCHUNK // IDXB, _win, 0)
    plsc.subcore_barrier()

    def _stage(st, carry):
        lo = pl.multiple_of(tstart + st * RSTG, RSTG)
        pltpu.sync_copy(acc.at[pl.ds(lo, RSTG)], stage)

        def _row(r, carry2):
            ndr = ndv[pl.ds(st * RSTG + r, 16)][0]
            nsr = nsv[pl.ds(st * RSTG + r, 16)][0] if relu_ns else None
            for g in range(8):
                sl = pl.ds(g * 16, 16)
                v = stage[r, sl] * ndr + bv[sl]
                if relu_ns:
                    v = jnp.maximum(v, 0.0) * nsr
                stage[r, sl] = v
            return carry2
        lax.fori_loop(0, RSTG, _row, 0)
        pltpu.sync_copy(stage, out.at[c, pl.ds(lo, RSTG)])
        return carry
    lax.fori_loop(0, RPT // RSTG, _stage, 0)


def _message_pass(tbl, sp, dp, nd, ns, bias, relu_ns):
    mesh = plsc.VectorSubcoreMesh(core_axis_name="c", subcore_axis_name="s")
    return pl.kernel(
        functools.partial(_msg_body, relu_ns),
        out_type=jax.ShapeDtypeStruct((2, NU_PAD, 128), _f32),
        mesh=mesh,
        scratch_types=[
            pltpu.VMEM_SHARED((NU_PAD, 128), _f32),
            pltpu.VMEM((CHUNK, 128), _f32),
            pltpu.VMEM((IDXB, CHUNK), _i32),
            pltpu.VMEM((IDXB, CHUNK), _i32),
            pltpu.VMEM((16, 128), _f32),
            pltpu.VMEM((RSTG, 128), _f32),
            pltpu.VMEM((RPT + 16,), _f32),
            pltpu.VMEM((RPT + 16,), _f32),
            pltpu.VMEM((128,), _f32),
            pltpu.SemaphoreType.DMA,
        ],
    )(tbl, sp, dp, nd, ns, bias)


_RB = 2000


def _tc1_body(xu, xs, wu, ws, dg, hu, hs, nrm):
    nb = lax.rsqrt(jnp.maximum(dg[...], 1.0))
    nrm[...] = nb
    hu[...] = jnp.dot(xu[...] * nb[:, 0:1], wu[...], preferred_element_type=_f32)
    hs[...] = jnp.dot(xs[...] * nb[:, 2:3], ws[...], preferred_element_type=_f32)


def _tc1(x_user, x_sub, w1u, w1s, degT):
    grid = (NU // _RB,)
    return pl.pallas_call(
        _tc1_body,
        grid=grid,
        in_specs=[
            pl.BlockSpec((_RB, D_IN), lambda i: (i, 0)),
            pl.BlockSpec((_RB, D_IN), lambda i: (i, 0)),
            pl.BlockSpec((D_IN, D_H), lambda i: (0, 0)),
            pl.BlockSpec((D_IN, D_H), lambda i: (0, 0)),
            pl.BlockSpec((_RB, 4), lambda i: (i, 0)),
        ],
        out_specs=[
            pl.BlockSpec((_RB, D_H), lambda i: (i, 0)),
            pl.BlockSpec((_RB, D_H), lambda i: (i, 0)),
            pl.BlockSpec((_RB, 4), lambda i: (i, 0)),
        ],
        out_shape=[
            jax.ShapeDtypeStruct((NU, D_H), _f32),
            jax.ShapeDtypeStruct((NU, D_H), _f32),
            jax.ShapeDtypeStruct((NU, 4), _f32),
        ],
    )(x_user, x_sub, w1u, w1s, degT)


_RB2 = 2048


def _tc2_body(hu2, hs2, wu, ws, gu, gs):
    gu[...] = (jnp.dot(hu2[0], wu[0:128, :], preferred_element_type=_f32)
               + jnp.dot(hu2[1], wu[128:256, :], preferred_element_type=_f32))
    gs[...] = (jnp.dot(hs2[0], ws[0:128, :], preferred_element_type=_f32)
               + jnp.dot(hs2[1], ws[128:256, :], preferred_element_type=_f32))


def _tc2(h_user_s, h_sub_s, w2u, w2s):
    grid = (NU_PAD // _RB2,)
    return pl.pallas_call(
        _tc2_body,
        grid=grid,
        in_specs=[
            pl.BlockSpec((2, _RB2, 128), lambda i: (0, i, 0)),
            pl.BlockSpec((2, _RB2, 128), lambda i: (0, i, 0)),
            pl.BlockSpec((D_H, D_OUT), lambda i: (0, 0)),
            pl.BlockSpec((D_H, D_OUT), lambda i: (0, 0)),
        ],
        out_specs=[
            pl.BlockSpec((_RB2, D_OUT), lambda i: (i, 0)),
            pl.BlockSpec((_RB2, D_OUT), lambda i: (i, 0)),
        ],
        out_shape=[
            jax.ShapeDtypeStruct((NU_PAD, D_OUT), _f32),
            jax.ShapeDtypeStruct((NU_PAD, D_OUT), _f32),
        ],
    )(h_user_s, h_sub_s, w2u, w2s)


def kernel(x_user, x_sub, edge_index_us, edge_index_su,
           W1_us, b1_us, W1_su, b1_su, W2_us, b2_us, W2_su, b2_su):
    sp_us = _pad_src(edge_index_us[0])
    dp_us = _pad_dst(edge_index_us[1])
    sp_su = _pad_src(edge_index_su[0])
    dp_su = _pad_dst(edge_index_su[1])

    sp12 = jnp.concatenate([sp_us, sp_su])
    dp12 = jnp.concatenate([dp_us, dp_su])
    degs = _degrees(sp12, dp12).reshape(4, ACC_H)
    degT = degs[:, :NU].T

    hu, hs, nrmT = _tc1(x_user, x_sub, W1_us, W1_su, degT)
    ns_us, nd_us = nrmT[:, 0], nrmT[:, 1]
    ns_su, nd_su = nrmT[:, 2], nrmT[:, 3]

    zpad = jnp.zeros((NPAD, D_H), _f32)
    hu_p = jnp.concatenate([hu, zpad])
    hs_p = jnp.concatenate([hs, zpad])
    tbl_us = jnp.stack([hu_p[:, :128], hu_p[:, 128:]])
    tbl_su = jnp.stack([hs_p[:, :128], hs_p[:, 128:]])

    npd = jnp.ones((NU_PAD - NU,), _f32)

    def _n2(a, b):
        return jnp.stack([jnp.concatenate([a, npd]),
                          jnp.concatenate([b, npd])]).reshape(2, 1, NU_PAD)

    h_sub_s = _message_pass(
        tbl_us, jnp.concatenate([sp_us, sp_us]), jnp.concatenate([dp_us, dp_us]),
        _n2(nd_us, nd_us), _n2(ns_su, ns_su),
        b1_us.reshape(2, 1, 128), relu_ns=True)
    h_user_s = _message_pass(
        tbl_su, jnp.concatenate([sp_su, sp_su]), jnp.concatenate([dp_su, dp_su]),
        _n2(nd_su, nd_su), _n2(ns_us, ns_us),
        b1_su.reshape(2, 1, 128), relu_ns=True)

    gu, gs = _tc2(h_user_s, h_sub_s, W2_us, W2_su)

    zpad2 = jnp.zeros((NPAD, D_OUT), _f32)
    tbl2 = jnp.stack([jnp.concatenate([gu[:NU], zpad2]),
                      jnp.concatenate([gs[:NU], zpad2])])
    outs = _message_pass(
        tbl2, sp12, dp12,
        _n2(nd_us, nd_su), _n2(nd_us, nd_su),
        jnp.stack([b2_us, b2_su]).reshape(2, 1, 128), relu_ns=False)

    return outs[1, :NU], outs[0, :NU]


# speedup vs baseline: 5.0212x; 1.2668x over previous
import functools

import jax
import jax.numpy as jnp
from jax import lax
from jax.experimental import pallas as pl
from jax.experimental.pallas import tpu as pltpu
from jax.experimental.pallas import tpu_sc as plsc

NU = 10000
E = 160000
D_IN = 256
D_H = 256
D_OUT = 128

NTILES = 16
CHUNK = 128
NCHUNK = 80
EPT = NCHUNK * CHUNK
E_PAD = NTILES * EPT
NPAD = E_PAD - E
TBL_N = NU + NPAD
NTRASH = 240
NU_PAD = 10240
ACC_H = 14336
ZCH_H = ACC_H // NTILES
RPT = NU_PAD // NTILES
RSTG = 64
IDXB = 8

_f32 = jnp.float32
_i32 = jnp.int32


def _pad_src(idx):
    pad = jnp.arange(NU, NU + NPAD, dtype=_i32)
    return jnp.concatenate([idx.astype(_i32), pad]).reshape(NTILES, NCHUNK, CHUNK)


def _pad_dst(idx):
    pad = NU + jnp.arange(NPAD, dtype=_i32) % NTRASH
    return jnp.concatenate([idx.astype(_i32), pad]).reshape(NTILES, NCHUNK, CHUNK)


def _deg_body(sp, dp, out, acc_s, acc_d, idx_s, idx_d, ones_v, zbuf):
    c = lax.axis_index("c")
    t = lax.axis_index("s")

    def _zi(i, carry):
        zbuf[pl.ds(i * 16, 16)] = jnp.zeros((16,), _f32)
        return carry
    lax.fori_loop(0, ZCH_H // 16, _zi, 0)

    def _oi(i, carry):
        ones_v[pl.ds(i * 16, 16)] = jnp.ones((16,), _f32)
        return carry
    lax.fori_loop(0, CHUNK // 16, _oi, 0)

    zoff = pl.multiple_of(t * ZCH_H, ZCH_H)
    pltpu.sync_copy(zbuf, acc_s.at[pl.ds(zoff, ZCH_H)])
    pltpu.sync_copy(zbuf, acc_d.at[pl.ds(zoff, ZCH_H)])
    w = c * NTILES + t
    pltpu.sync_copy(sp.at[w], idx_s)
    pltpu.sync_copy(dp.at[w], idx_d)
    plsc.subcore_barrier()

    def _ch(j, carry):
        pltpu.sync_copy(ones_v, acc_s.at[idx_s.at[j]], add=True)
        pltpu.sync_copy(ones_v, acc_d.at[idx_d.at[j]], add=True)
        return carry
    lax.fori_loop(0, NCHUNK, _ch, 0)
    plsc.subcore_barrier()

    pltpu.sync_copy(acc_s.at[pl.ds(zoff, ZCH_H)], out.at[2 * c, 0, pl.ds(zoff, ZCH_H)])
    pltpu.sync_copy(acc_d.at[pl.ds(zoff, ZCH_H)], out.at[2 * c + 1, 0, pl.ds(zoff, ZCH_H)])


def _degrees(sp12, dp12):
    mesh = plsc.VectorSubcoreMesh(core_axis_name="c", subcore_axis_name="s")
    return pl.kernel(
        _deg_body,
        out_type=jax.ShapeDtypeStruct((4, 1, ACC_H), _f32),
        mesh=mesh,
        scratch_types=[
            pltpu.VMEM_SHARED((ACC_H,), _f32),
            pltpu.VMEM_SHARED((ACC_H,), _f32),
            pltpu.VMEM((NCHUNK, CHUNK), _i32),
            pltpu.VMEM((NCHUNK, CHUNK), _i32),
            pltpu.VMEM((CHUNK,), _f32),
            pltpu.VMEM((ZCH_H,), _f32),
        ],
    )(sp12, dp12)


def _msg_body(relu_ns, tbl, sp, dp, nd, ns, bias, out,
              acc, rows_a, rows_b, idx_s, idx_d, zbuf, stage, ndv, nsv, bv,
              sem_ga, sem_gb, sem_sa, sem_sb):
    c = lax.axis_index("c")
    t = lax.axis_index("s")
    w = c * NTILES + t

    def _zi(i, carry):
        for g in range(8):
            zbuf[i, pl.ds(g * 16, 16)] = jnp.zeros((16,), _f32)
        return carry
    lax.fori_loop(0, 16, _zi, 0)

    def _za(k, carry):
        off = pl.multiple_of(t * RPT + k * 16, 16)
        pltpu.sync_copy(zbuf, acc.at[pl.ds(off, 16)])
        return carry
    lax.fori_loop(0, RPT // 16, _za, 0)

    tstart = pl.multiple_of(t * RPT, RPT)
    pltpu.sync_copy(nd.at[c, 0, pl.ds(tstart, RPT)], ndv.at[pl.ds(0, RPT)])
    if relu_ns:
        pltpu.sync_copy(ns.at[c, 0, pl.ds(tstart, RPT)], nsv.at[pl.ds(0, RPT)])
    pltpu.sync_copy(bias.at[c, 0], bv)
    plsc.subcore_barrier()

    bufs = (rows_a, rows_b)
    gsem = (sem_ga, sem_gb)
    ssem = (sem_sa, sem_sb)

    def _win(jw, carry):
        woff = pl.multiple_of(jw * IDXB, IDXB)
        pltpu.sync_copy(sp.at[w, pl.ds(woff, IDXB)], idx_s)
        pltpu.sync_copy(dp.at[w, pl.ds(woff, IDXB)], idx_d)

        gd = [None, None]
        sd = [None, None]
        for k in range(2):
            gd[k] = pltpu.async_copy(tbl.at[c].at[idx_s.at[k]], bufs[k], gsem[k])
        for k in range(IDXB):
            b = k % 2
            gd[b].wait()
            sd[b] = pltpu.async_copy(bufs[b], acc.at[idx_d.at[k]], ssem[b],
                                     add=True)
            if k + 2 < IDXB:
                sd[b].wait()
                gd[b] = pltpu.async_copy(tbl.at[c].at[idx_s.at[k + 2]],
                                         bufs[b], gsem[b])
        sd[0].wait()
        sd[1].wait()
        return carry
    lax.fori_loop(0, NCHUNK // IDXB, _win, 0)
    plsc.subcore_barrier()

    def _stage(st, carry):
        lo = pl.multiple_of(tstart + st * RSTG, RSTG)
        pltpu.sync_copy(acc.at[pl.ds(lo, RSTG)], stage)

        def _row(r, carry2):
            ndr = ndv[pl.ds(st * RSTG + r, 16)][0]
            nsr = nsv[pl.ds(st * RSTG + r, 16)][0] if relu_ns else None
            for g in range(8):
                sl = pl.ds(g * 16, 16)
                v = stage[r, sl] * ndr + bv[sl]
                if relu_ns:
                    v = jnp.maximum(v, 0.0) * nsr
                stage[r, sl] = v
            return carry2
        lax.fori_loop(0, RSTG, _row, 0)
        pltpu.sync_copy(stage, out.at[c, pl.ds(lo, RSTG)])
        return carry
    lax.fori_loop(0, RPT // RSTG, _stage, 0)


def _message_pass(tbl, sp, dp, nd, ns, bias, relu_ns):
    mesh = plsc.VectorSubcoreMesh(core_axis_name="c", subcore_axis_name="s")
    return pl.kernel(
        functools.partial(_msg_body, relu_ns),
        out_type=jax.ShapeDtypeStruct((2, NU_PAD, 128), _f32),
        mesh=mesh,
        scratch_types=[
            pltpu.VMEM_SHARED((NU_PAD, 128), _f32),
            pltpu.VMEM((CHUNK, 128), _f32),
            pltpu.VMEM((CHUNK, 128), _f32),
            pltpu.VMEM((IDXB, CHUNK), _i32),
            pltpu.VMEM((IDXB, CHUNK), _i32),
            pltpu.VMEM((16, 128), _f32),
            pltpu.VMEM((RSTG, 128), _f32),
            pltpu.VMEM((RPT + 16,), _f32),
            pltpu.VMEM((RPT + 16,), _f32),
            pltpu.VMEM((128,), _f32),
            pltpu.SemaphoreType.DMA,
            pltpu.SemaphoreType.DMA,
            pltpu.SemaphoreType.DMA,
            pltpu.SemaphoreType.DMA,
        ],
    )(tbl, sp, dp, nd, ns, bias)


_RB = 2000


def _tc1_body(xu, xs, wu, ws, dg, hu, hs, nrm):
    nb = lax.rsqrt(jnp.maximum(dg[...], 1.0))
    nrm[...] = nb
    hu[...] = jnp.dot(xu[...] * nb[:, 0:1], wu[...], preferred_element_type=_f32)
    hs[...] = jnp.dot(xs[...] * nb[:, 2:3], ws[...], preferred_element_type=_f32)


def _tc1(x_user, x_sub, w1u, w1s, degT):
    grid = (NU // _RB,)
    return pl.pallas_call(
        _tc1_body,
        grid=grid,
        in_specs=[
            pl.BlockSpec((_RB, D_IN), lambda i: (i, 0)),
            pl.BlockSpec((_RB, D_IN), lambda i: (i, 0)),
            pl.BlockSpec((D_IN, D_H), lambda i: (0, 0)),
            pl.BlockSpec((D_IN, D_H), lambda i: (0, 0)),
            pl.BlockSpec((_RB, 4), lambda i: (i, 0)),
        ],
        out_specs=[
            pl.BlockSpec((_RB, D_H), lambda i: (i, 0)),
            pl.BlockSpec((_RB, D_H), lambda i: (i, 0)),
            pl.BlockSpec((_RB, 4), lambda i: (i, 0)),
        ],
        out_shape=[
            jax.ShapeDtypeStruct((NU, D_H), _f32),
            jax.ShapeDtypeStruct((NU, D_H), _f32),
            jax.ShapeDtypeStruct((NU, 4), _f32),
        ],
    )(x_user, x_sub, w1u, w1s, degT)


_RB2 = 2048


def _tc2_body(hu2, hs2, wu, ws, gu, gs):
    gu[...] = (jnp.dot(hu2[0], wu[0:128, :], preferred_element_type=_f32)
               + jnp.dot(hu2[1], wu[128:256, :], preferred_element_type=_f32))
    gs[...] = (jnp.dot(hs2[0], ws[0:128, :], preferred_element_type=_f32)
               + jnp.dot(hs2[1], ws[128:256, :], preferred_element_type=_f32))


def _tc2(h_user_s, h_sub_s, w2u, w2s):
    grid = (NU_PAD // _RB2,)
    return pl.pallas_call(
        _tc2_body,
        grid=grid,
        in_specs=[
            pl.BlockSpec((2, _RB2, 128), lambda i: (0, i, 0)),
            pl.BlockSpec((2, _RB2, 128), lambda i: (0, i, 0)),
            pl.BlockSpec((D_H, D_OUT), lambda i: (0, 0)),
            pl.BlockSpec((D_H, D_OUT), lambda i: (0, 0)),
        ],
        out_specs=[
            pl.BlockSpec((_RB2, D_OUT), lambda i: (i, 0)),
            pl.BlockSpec((_RB2, D_OUT), lambda i: (i, 0)),
        ],
        out_shape=[
            jax.ShapeDtypeStruct((NU_PAD, D_OUT), _f32),
            jax.ShapeDtypeStruct((NU_PAD, D_OUT), _f32),
        ],
    )(h_user_s, h_sub_s, w2u, w2s)


def kernel(x_user, x_sub, edge_index_us, edge_index_su,
           W1_us, b1_us, W1_su, b1_su, W2_us, b2_us, W2_su, b2_su):
    sp_us = _pad_src(edge_index_us[0])
    dp_us = _pad_dst(edge_index_us[1])
    sp_su = _pad_src(edge_index_su[0])
    dp_su = _pad_dst(edge_index_su[1])

    sp12 = jnp.concatenate([sp_us, sp_su])
    dp12 = jnp.concatenate([dp_us, dp_su])
    degs = _degrees(sp12, dp12).reshape(4, ACC_H)
    degT = degs[:, :NU].T

    hu, hs, nrmT = _tc1(x_user, x_sub, W1_us, W1_su, degT)
    ns_us, nd_us = nrmT[:, 0], nrmT[:, 1]
    ns_su, nd_su = nrmT[:, 2], nrmT[:, 3]

    zpad = jnp.zeros((NPAD, D_H), _f32)
    hu_p = jnp.concatenate([hu, zpad])
    hs_p = jnp.concatenate([hs, zpad])
    tbl_us = jnp.stack([hu_p[:, :128], hu_p[:, 128:]])
    tbl_su = jnp.stack([hs_p[:, :128], hs_p[:, 128:]])

    npd = jnp.ones((NU_PAD - NU,), _f32)

    def _n2(a, b):
        return jnp.stack([jnp.concatenate([a, npd]),
                          jnp.concatenate([b, npd])]).reshape(2, 1, NU_PAD)

    h_sub_s = _message_pass(
        tbl_us, jnp.concatenate([sp_us, sp_us]), jnp.concatenate([dp_us, dp_us]),
        _n2(nd_us, nd_us), _n2(ns_su, ns_su),
        b1_us.reshape(2, 1, 128), relu_ns=True)
    h_user_s = _message_pass(
        tbl_su, jnp.concatenate([sp_su, sp_su]), jnp.concatenate([dp_su, dp_su]),
        _n2(nd_su, nd_su), _n2(ns_us, ns_us),
        b1_su.reshape(2, 1, 128), relu_ns=True)

    gu, gs = _tc2(h_user_s, h_sub_s, W2_us, W2_su)

    zpad2 = jnp.zeros((NPAD, D_OUT), _f32)
    tbl2 = jnp.stack([jnp.concatenate([gu[:NU], zpad2]),
                      jnp.concatenate([gs[:NU], zpad2])])
    outs = _message_pass(
        tbl2, sp12, dp12,
        _n2(nd_us, nd_su), _n2(nd_us, nd_su),
        jnp.stack([b2_us, b2_su]).reshape(2, 1, 128), relu_ns=False)

    return outs[1, :NU], outs[0, :NU]


# speedup vs baseline: 5.3087x; 1.0573x over previous
import functools

import jax
import jax.numpy as jnp
from jax import lax
from jax.experimental import pallas as pl
from jax.experimental.pallas import tpu as pltpu
from jax.experimental.pallas import tpu_sc as plsc

NU = 10000
E = 160000
D_IN = 256
D_H = 256
D_OUT = 128

NTILES = 16
CHUNK = 128
NCHUNK = 80
EPT = NCHUNK * CHUNK
E_PAD = NTILES * EPT
NPAD = E_PAD - E
TBL_N = NU + NPAD
NTRASH = 240
NU_PAD = 10240
ACC_H = 14336
ZCH_H = ACC_H // NTILES
RPT = NU_PAD // NTILES
RSTG = 64
IDXB = 16

_f32 = jnp.float32
_i32 = jnp.int32


def _pad_src(idx):
    pad = jnp.arange(NU, NU + NPAD, dtype=_i32)
    return jnp.concatenate([idx.astype(_i32), pad]).reshape(NTILES, NCHUNK, CHUNK)


def _pad_dst(idx):
    pad = NU + jnp.arange(NPAD, dtype=_i32) % NTRASH
    return jnp.concatenate([idx.astype(_i32), pad]).reshape(NTILES, NCHUNK, CHUNK)


def _deg_body(sp, dp, out, acc_s, acc_d, idx_s, idx_d, ones_v, zbuf):
    c = lax.axis_index("c")
    t = lax.axis_index("s")

    def _zi(i, carry):
        zbuf[pl.ds(i * 16, 16)] = jnp.zeros((16,), _f32)
        return carry
    lax.fori_loop(0, ZCH_H // 16, _zi, 0)

    def _oi(i, carry):
        ones_v[pl.ds(i * 16, 16)] = jnp.ones((16,), _f32)
        return carry
    lax.fori_loop(0, CHUNK // 16, _oi, 0)

    zoff = pl.multiple_of(t * ZCH_H, ZCH_H)
    pltpu.sync_copy(zbuf, acc_s.at[pl.ds(zoff, ZCH_H)])
    pltpu.sync_copy(zbuf, acc_d.at[pl.ds(zoff, ZCH_H)])
    w = c * NTILES + t
    pltpu.sync_copy(sp.at[w], idx_s)
    pltpu.sync_copy(dp.at[w], idx_d)
    plsc.subcore_barrier()

    def _ch(j, carry):
        pltpu.sync_copy(ones_v, acc_s.at[idx_s.at[j]], add=True)
        pltpu.sync_copy(ones_v, acc_d.at[idx_d.at[j]], add=True)
        return carry
    lax.fori_loop(0, NCHUNK, _ch, 0)
    plsc.subcore_barrier()

    pltpu.sync_copy(acc_s.at[pl.ds(zoff, ZCH_H)], out.at[2 * c, 0, pl.ds(zoff, ZCH_H)])
    pltpu.sync_copy(acc_d.at[pl.ds(zoff, ZCH_H)], out.at[2 * c + 1, 0, pl.ds(zoff, ZCH_H)])


def _degrees(sp12, dp12):
    mesh = plsc.VectorSubcoreMesh(core_axis_name="c", subcore_axis_name="s")
    return pl.kernel(
        _deg_body,
        out_type=jax.ShapeDtypeStruct((4, 1, ACC_H), _f32),
        mesh=mesh,
        scratch_types=[
            pltpu.VMEM_SHARED((ACC_H,), _f32),
            pltpu.VMEM_SHARED((ACC_H,), _f32),
            pltpu.VMEM((NCHUNK, CHUNK), _i32),
            pltpu.VMEM((NCHUNK, CHUNK), _i32),
            pltpu.VMEM((CHUNK,), _f32),
            pltpu.VMEM((ZCH_H,), _f32),
        ],
    )(sp12, dp12)


def _msg_body(relu_ns, tbl, sp, dp, nd, ns, bias, out,
              acc, rows_a, rows_b, idx_s, idx_d, zbuf, stage, ndv, nsv, bv,
              sem_ga, sem_gb, sem_sa, sem_sb):
    c = lax.axis_index("c")
    t = lax.axis_index("s")
    w = c * NTILES + t

    def _zi(i, carry):
        for g in range(8):
            zbuf[i, pl.ds(g * 16, 16)] = jnp.zeros((16,), _f32)
        return carry
    lax.fori_loop(0, 16, _zi, 0)

    def _za(k, carry):
        off = pl.multiple_of(t * RPT + k * 16, 16)
        pltpu.sync_copy(zbuf, acc.at[pl.ds(off, 16)])
        return carry
    lax.fori_loop(0, RPT // 16, _za, 0)

    tstart = pl.multiple_of(t * RPT, RPT)
    pltpu.sync_copy(nd.at[c, 0, pl.ds(tstart, RPT)], ndv.at[pl.ds(0, RPT)])
    if relu_ns:
        pltpu.sync_copy(ns.at[c, 0, pl.ds(tstart, RPT)], nsv.at[pl.ds(0, RPT)])
    pltpu.sync_copy(bias.at[c, 0], bv)
    plsc.subcore_barrier()

    bufs = (rows_a, rows_b)
    gsem = (sem_ga, sem_gb)
    ssem = (sem_sa, sem_sb)

    def _win(jw, carry):
        woff = pl.multiple_of(jw * IDXB, IDXB)
        pltpu.sync_copy(sp.at[w, pl.ds(woff, IDXB)], idx_s)
        pltpu.sync_copy(dp.at[w, pl.ds(woff, IDXB)], idx_d)

        gd = [None, None]
        sd = [None, None]
        for k in range(2):
            gd[k] = pltpu.async_copy(tbl.at[c].at[idx_s.at[k]], bufs[k], gsem[k])
        for k in range(IDXB):
            b = k % 2
            gd[b].wait()
            sd[b] = pltpu.async_copy(bufs[b], acc.at[idx_d.at[k]], ssem[b],
                                     add=True)
            if k + 2 < IDXB:
                sd[b].wait()
                gd[b] = pltpu.async_copy(tbl.at[c].at[idx_s.at[k + 2]],
                                         bufs[b], gsem[b])
        sd[0].wait()
        sd[1].wait()
        return carry
    lax.fori_loop(0, NCHUNK // IDXB, _win, 0)
    plsc.subcore_barrier()

    def _stage(st, carry):
        lo = pl.multiple_of(tstart + st * RSTG, RSTG)
        pltpu.sync_copy(acc.at[pl.ds(lo, RSTG)], stage)

        def _row(r, carry2):
            ndr = ndv[pl.ds(st * RSTG + r, 16)][0]
            nsr = nsv[pl.ds(st * RSTG + r, 16)][0] if relu_ns else None
            for g in range(8):
                sl = pl.ds(g * 16, 16)
                v = stage[r, sl] * ndr + bv[sl]
                if relu_ns:
                    v = jnp.maximum(v, 0.0) * nsr
                stage[r, sl] = v
            return carry2
        lax.fori_loop(0, RSTG, _row, 0)
        pltpu.sync_copy(stage, out.at[c, pl.ds(lo, RSTG)])
        return carry
    lax.fori_loop(0, RPT // RSTG, _stage, 0)


def _message_pass(tbl, sp, dp, nd, ns, bias, relu_ns):
    mesh = plsc.VectorSubcoreMesh(core_axis_name="c", subcore_axis_name="s")
    return pl.kernel(
        functools.partial(_msg_body, relu_ns),
        out_type=jax.ShapeDtypeStruct((2, NU_PAD, 128), _f32),
        mesh=mesh,
        scratch_types=[
            pltpu.VMEM_SHARED((NU_PAD, 128), _f32),
            pltpu.VMEM((CHUNK, 128), _f32),
            pltpu.VMEM((CHUNK, 128), _f32),
            pltpu.VMEM((IDXB, CHUNK), _i32),
            pltpu.VMEM((IDXB, CHUNK), _i32),
            pltpu.VMEM((16, 128), _f32),
            pltpu.VMEM((RSTG, 128), _f32),
            pltpu.VMEM((RPT + 16,), _f32),
            pltpu.VMEM((RPT + 16,), _f32),
            pltpu.VMEM((128,), _f32),
            pltpu.SemaphoreType.DMA,
            pltpu.SemaphoreType.DMA,
            pltpu.SemaphoreType.DMA,
            pltpu.SemaphoreType.DMA,
        ],
    )(tbl, sp, dp, nd, ns, bias)


_RB = 2000


def _tc1_body(xu, xs, wu, ws, dg, hu, hs, nrm):
    nb = lax.rsqrt(jnp.maximum(dg[...], 1.0))
    nrm[...] = nb
    hu[...] = jnp.dot(xu[...] * nb[:, 0:1], wu[...], preferred_element_type=_f32)
    hs[...] = jnp.dot(xs[...] * nb[:, 2:3], ws[...], preferred_element_type=_f32)


def _tc1(x_user, x_sub, w1u, w1s, degT):
    grid = (NU // _RB,)
    return pl.pallas_call(
        _tc1_body,
        grid=grid,
        in_specs=[
            pl.BlockSpec((_RB, D_IN), lambda i: (i, 0)),
            pl.BlockSpec((_RB, D_IN), lambda i: (i, 0)),
            pl.BlockSpec((D_IN, D_H), lambda i: (0, 0)),
            pl.BlockSpec((D_IN, D_H), lambda i: (0, 0)),
            pl.BlockSpec((_RB, 4), lambda i: (i, 0)),
        ],
        out_specs=[
            pl.BlockSpec((_RB, D_H), lambda i: (i, 0)),
            pl.BlockSpec((_RB, D_H), lambda i: (i, 0)),
            pl.BlockSpec((_RB, 4), lambda i: (i, 0)),
        ],
        out_shape=[
            jax.ShapeDtypeStruct((NU, D_H), _f32),
            jax.ShapeDtypeStruct((NU, D_H), _f32),
            jax.ShapeDtypeStruct((NU, 4), _f32),
        ],
    )(x_user, x_sub, w1u, w1s, degT)


_RB2 = 2048


def _tc2_body(hu2, hs2, wu, ws, gu, gs):
    gu[...] = (jnp.dot(hu2[0], wu[0:128, :], preferred_element_type=_f32)
               + jnp.dot(hu2[1], wu[128:256, :], preferred_element_type=_f32))
    gs[...] = (jnp.dot(hs2[0], ws[0:128, :], preferred_element_type=_f32)
               + jnp.dot(hs2[1], ws[128:256, :], preferred_element_type=_f32))


def _tc2(h_user_s, h_sub_s, w2u, w2s):
    grid = (NU_PAD // _RB2,)
    return pl.pallas_call(
        _tc2_body,
        grid=grid,
        in_specs=[
            pl.BlockSpec((2, _RB2, 128), lambda i: (0, i, 0)),
            pl.BlockSpec((2, _RB2, 128), lambda i: (0, i, 0)),
            pl.BlockSpec((D_H, D_OUT), lambda i: (0, 0)),
            pl.BlockSpec((D_H, D_OUT), lambda i: (0, 0)),
        ],
        out_specs=[
            pl.BlockSpec((_RB2, D_OUT), lambda i: (i, 0)),
            pl.BlockSpec((_RB2, D_OUT), lambda i: (i, 0)),
        ],
        out_shape=[
            jax.ShapeDtypeStruct((NU_PAD, D_OUT), _f32),
            jax.ShapeDtypeStruct((NU_PAD, D_OUT), _f32),
        ],
    )(h_user_s, h_sub_s, w2u, w2s)


def kernel(x_user, x_sub, edge_index_us, edge_index_su,
           W1_us, b1_us, W1_su, b1_su, W2_us, b2_us, W2_su, b2_su):
    sp_us = _pad_src(edge_index_us[0])
    dp_us = _pad_dst(edge_index_us[1])
    sp_su = _pad_src(edge_index_su[0])
    dp_su = _pad_dst(edge_index_su[1])

    sp12 = jnp.concatenate([sp_us, sp_su])
    dp12 = jnp.concatenate([dp_us, dp_su])
    degs = _degrees(sp12, dp12).reshape(4, ACC_H)
    degT = degs[:, :NU].T

    hu, hs, nrmT = _tc1(x_user, x_sub, W1_us, W1_su, degT)
    ns_us, nd_us = nrmT[:, 0], nrmT[:, 1]
    ns_su, nd_su = nrmT[:, 2], nrmT[:, 3]

    zpad = jnp.zeros((NPAD, D_H), _f32)
    hu_p = jnp.concatenate([hu, zpad])
    hs_p = jnp.concatenate([hs, zpad])
    tbl_us = jnp.stack([hu_p[:, :128], hu_p[:, 128:]])
    tbl_su = jnp.stack([hs_p[:, :128], hs_p[:, 128:]])

    npd = jnp.ones((NU_PAD - NU,), _f32)

    def _n2(a, b):
        return jnp.stack([jnp.concatenate([a, npd]),
                          jnp.concatenate([b, npd])]).reshape(2, 1, NU_PAD)

    h_sub_s = _message_pass(
        tbl_us, jnp.concatenate([sp_us, sp_us]), jnp.concatenate([dp_us, dp_us]),
        _n2(nd_us, nd_us), _n2(ns_su, ns_su),
        b1_us.reshape(2, 1, 128), relu_ns=True)
    h_user_s = _message_pass(
        tbl_su, jnp.concatenate([sp_su, sp_su]), jnp.concatenate([dp_su, dp_su]),
        _n2(nd_su, nd_su), _n2(ns_us, ns_us),
        b1_su.reshape(2, 1, 128), relu_ns=True)

    gu, gs = _tc2(h_user_s, h_sub_s, W2_us, W2_su)

    zpad2 = jnp.zeros((NPAD, D_OUT), _f32)
    tbl2 = jnp.stack([jnp.concatenate([gu[:NU], zpad2]),
                      jnp.concatenate([gs[:NU], zpad2])])
    outs = _message_pass(
        tbl2, sp12, dp12,
        _n2(nd_us, nd_su), _n2(nd_us, nd_su),
        jnp.stack([b2_us, b2_su]).reshape(2, 1, 128), relu_ns=False)

    return outs[1, :NU], outs[0, :NU]


# speedup vs baseline: 5.5161x; 1.0391x over previous
import functools

import jax
import jax.numpy as jnp
from jax import lax
from jax.experimental import pallas as pl
from jax.experimental.pallas import tpu as pltpu
from jax.experimental.pallas import tpu_sc as plsc

NU = 10000
E = 160000
D_IN = 256
D_H = 256
D_OUT = 128

NTILES = 16
CHUNK = 128
NCHUNK = 80
EPT = NCHUNK * CHUNK
E_PAD = NTILES * EPT
NPAD = E_PAD - E
TBL_N = 12288
NTRASH = 240
NU_PAD = 10240
ACC_H = 14336
ZCH_H = ACC_H // NTILES
RPT = NU_PAD // NTILES
RSTG = 64
IDXB = 16

_f32 = jnp.float32
_i32 = jnp.int32


def _pad_src(idx):
    pad = NU + jnp.arange(NPAD, dtype=_i32) % (TBL_N - NU)
    return jnp.concatenate([idx.astype(_i32), pad]).reshape(NTILES, NCHUNK, CHUNK)


def _pad_dst(idx):
    pad = NU + jnp.arange(NPAD, dtype=_i32) % NTRASH
    return jnp.concatenate([idx.astype(_i32), pad]).reshape(NTILES, NCHUNK, CHUNK)


def _deg_body(sp, dp, out, acc_s, acc_d, idx_s, idx_d, ones_v, zbuf):
    c = lax.axis_index("c")
    t = lax.axis_index("s")

    def _zi(i, carry):
        zbuf[pl.ds(i * 16, 16)] = jnp.zeros((16,), _f32)
        return carry
    lax.fori_loop(0, ZCH_H // 16, _zi, 0)

    def _oi(i, carry):
        ones_v[pl.ds(i * 16, 16)] = jnp.ones((16,), _f32)
        return carry
    lax.fori_loop(0, CHUNK // 16, _oi, 0)

    zoff = pl.multiple_of(t * ZCH_H, ZCH_H)
    pltpu.sync_copy(zbuf, acc_s.at[pl.ds(zoff, ZCH_H)])
    pltpu.sync_copy(zbuf, acc_d.at[pl.ds(zoff, ZCH_H)])
    w = c * NTILES + t
    pltpu.sync_copy(sp.at[w], idx_s)
    pltpu.sync_copy(dp.at[w], idx_d)
    plsc.subcore_barrier()

    def _ch(j, carry):
        pltpu.sync_copy(ones_v, acc_s.at[idx_s.at[j]], add=True)
        pltpu.sync_copy(ones_v, acc_d.at[idx_d.at[j]], add=True)
        return carry
    lax.fori_loop(0, NCHUNK, _ch, 0)
    plsc.subcore_barrier()

    pltpu.sync_copy(acc_s.at[pl.ds(zoff, ZCH_H)], out.at[2 * c, 0, pl.ds(zoff, ZCH_H)])
    pltpu.sync_copy(acc_d.at[pl.ds(zoff, ZCH_H)], out.at[2 * c + 1, 0, pl.ds(zoff, ZCH_H)])


def _degrees(sp12, dp12):
    mesh = plsc.VectorSubcoreMesh(core_axis_name="c", subcore_axis_name="s")
    return pl.kernel(
        _deg_body,
        out_type=jax.ShapeDtypeStruct((4, 1, ACC_H), _f32),
        mesh=mesh,
        scratch_types=[
            pltpu.VMEM_SHARED((ACC_H,), _f32),
            pltpu.VMEM_SHARED((ACC_H,), _f32),
            pltpu.VMEM((NCHUNK, CHUNK), _i32),
            pltpu.VMEM((NCHUNK, CHUNK), _i32),
            pltpu.VMEM((CHUNK,), _f32),
            pltpu.VMEM((ZCH_H,), _f32),
        ],
    )(sp12, dp12)


def _msg_body(relu_ns, tbl, sp, dp, nd, ns, bias, out,
              acc, rows_a, rows_b, idx_s, idx_d, zbuf, stage, ndv, nsv, bv,
              sem_ga, sem_gb, sem_sa, sem_sb):
    c = lax.axis_index("c")
    t = lax.axis_index("s")
    w = c * NTILES + t

    def _zi(i, carry):
        for g in range(8):
            zbuf[i, pl.ds(g * 16, 16)] = jnp.zeros((16,), _f32)
        return carry
    lax.fori_loop(0, 16, _zi, 0)

    def _za(k, carry):
        off = pl.multiple_of(t * RPT + k * 16, 16)
        pltpu.sync_copy(zbuf, acc.at[pl.ds(off, 16)])
        return carry
    lax.fori_loop(0, RPT // 16, _za, 0)

    tstart = pl.multiple_of(t * RPT, RPT)
    pltpu.sync_copy(nd.at[c, 0, pl.ds(tstart, RPT)], ndv.at[pl.ds(0, RPT)])
    if relu_ns:
        pltpu.sync_copy(ns.at[c, 0, pl.ds(tstart, RPT)], nsv.at[pl.ds(0, RPT)])
    pltpu.sync_copy(bias.at[c, 0], bv)
    plsc.subcore_barrier()

    bufs = (rows_a, rows_b)
    gsem = (sem_ga, sem_gb)
    ssem = (sem_sa, sem_sb)

    def _win(jw, carry):
        woff = pl.multiple_of(jw * IDXB, IDXB)
        pltpu.sync_copy(sp.at[w, pl.ds(woff, IDXB)], idx_s)
        pltpu.sync_copy(dp.at[w, pl.ds(woff, IDXB)], idx_d)

        gd = [None, None]
        sd = [None, None]
        for k in range(2):
            gd[k] = pltpu.async_copy(tbl.at[c].at[idx_s.at[k]], bufs[k], gsem[k])
        for k in range(IDXB):
            b = k % 2
            gd[b].wait()
            sd[b] = pltpu.async_copy(bufs[b], acc.at[idx_d.at[k]], ssem[b],
                                     add=True)
            if k + 2 < IDXB:
                sd[b].wait()
                gd[b] = pltpu.async_copy(tbl.at[c].at[idx_s.at[k + 2]],
                                         bufs[b], gsem[b])
        sd[0].wait()
        sd[1].wait()
        return carry
    lax.fori_loop(0, NCHUNK // IDXB, _win, 0)
    plsc.subcore_barrier()

    def _stage(st, carry):
        lo = pl.multiple_of(tstart + st * RSTG, RSTG)
        pltpu.sync_copy(acc.at[pl.ds(lo, RSTG)], stage)

        def _row(r, carry2):
            ndr = ndv[pl.ds(st * RSTG + r, 16)][0]
            nsr = nsv[pl.ds(st * RSTG + r, 16)][0] if relu_ns else None
            for g in range(8):
                sl = pl.ds(g * 16, 16)
                v = stage[r, sl] * ndr + bv[sl]
                if relu_ns:
                    v = jnp.maximum(v, 0.0) * nsr
                stage[r, sl] = v
            return carry2
        lax.fori_loop(0, RSTG, _row, 0)
        pltpu.sync_copy(stage, out.at[c, pl.ds(lo, RSTG)])
        return carry
    lax.fori_loop(0, RPT // RSTG, _stage, 0)


def _message_pass(tbl, sp, dp, nd, ns, bias, relu_ns):
    mesh = plsc.VectorSubcoreMesh(core_axis_name="c", subcore_axis_name="s")
    return pl.kernel(
        functools.partial(_msg_body, relu_ns),
        out_type=jax.ShapeDtypeStruct((2, NU_PAD, 128), _f32),
        mesh=mesh,
        scratch_types=[
            pltpu.VMEM_SHARED((NU_PAD, 128), _f32),
            pltpu.VMEM((CHUNK, 128), _f32),
            pltpu.VMEM((CHUNK, 128), _f32),
            pltpu.VMEM((IDXB, CHUNK), _i32),
            pltpu.VMEM((IDXB, CHUNK), _i32),
            pltpu.VMEM((16, 128), _f32),
            pltpu.VMEM((RSTG, 128), _f32),
            pltpu.VMEM((RPT + 16,), _f32),
            pltpu.VMEM((RPT + 16,), _f32),
            pltpu.VMEM((128,), _f32),
            pltpu.SemaphoreType.DMA,
            pltpu.SemaphoreType.DMA,
            pltpu.SemaphoreType.DMA,
            pltpu.SemaphoreType.DMA,
        ],
    )(tbl, sp, dp, nd, ns, bias)


_RB = 2048


def _tc1_body(xu, xs, wu, ws, dg, tbu, tbs, nrm):
    nb = lax.rsqrt(jnp.maximum(dg[...], 1.0))
    nrm[...] = nb
    hu = jnp.dot(xu[...] * nb[:, 0:1], wu[...], preferred_element_type=_f32)
    hs = jnp.dot(xs[...] * nb[:, 2:3], ws[...], preferred_element_type=_f32)
    tbu[...] = jnp.stack([hu[:, 0:128], hu[:, 128:256]])
    tbs[...] = jnp.stack([hs[:, 0:128], hs[:, 128:256]])


def _tc1(x_user, x_sub, w1u, w1s, degT):
    return pl.pallas_call(
        _tc1_body,
        grid=(NU_PAD // _RB,),
        in_specs=[
            pl.BlockSpec((_RB, D_IN), lambda i: (i, 0)),
            pl.BlockSpec((_RB, D_IN), lambda i: (i, 0)),
            pl.BlockSpec((D_IN, D_H), lambda i: (0, 0)),
            pl.BlockSpec((D_IN, D_H), lambda i: (0, 0)),
            pl.BlockSpec((_RB, 4), lambda i: (i, 0)),
        ],
        out_specs=[
            pl.BlockSpec((2, _RB, 128), lambda i: (0, i, 0)),
            pl.BlockSpec((2, _RB, 128), lambda i: (0, i, 0)),
            pl.BlockSpec((_RB, 4), lambda i: (i, 0)),
        ],
        out_shape=[
            jax.ShapeDtypeStruct((2, TBL_N, 128), _f32),
            jax.ShapeDtypeStruct((2, TBL_N, 128), _f32),
            jax.ShapeDtypeStruct((NU_PAD, 4), _f32),
        ],
    )(x_user, x_sub, w1u, w1s, degT)


def _tc2_body(hu2, hs2, wu, ws, tb2):
    gu = (jnp.dot(hu2[0], wu[0:128, :], preferred_element_type=_f32)
          + jnp.dot(hu2[1], wu[128:256, :], preferred_element_type=_f32))
    gs = (jnp.dot(hs2[0], ws[0:128, :], preferred_element_type=_f32)
          + jnp.dot(hs2[1], ws[128:256, :], preferred_element_type=_f32))
    tb2[...] = jnp.stack([gu, gs])


def _tc2(h_user_s, h_sub_s, w2u, w2s):
    return pl.pallas_call(
        _tc2_body,
        grid=(NU_PAD // _RB,),
        in_specs=[
            pl.BlockSpec((2, _RB, 128), lambda i: (0, i, 0)),
            pl.BlockSpec((2, _RB, 128), lambda i: (0, i, 0)),
            pl.BlockSpec((D_H, D_OUT), lambda i: (0, 0)),
            pl.BlockSpec((D_H, D_OUT), lambda i: (0, 0)),
        ],
        out_specs=[
            pl.BlockSpec((2, _RB, 128), lambda i: (0, i, 0)),
        ],
        out_shape=[
            jax.ShapeDtypeStruct((2, TBL_N, 128), _f32),
        ],
    )(h_user_s, h_sub_s, w2u, w2s)


def kernel(x_user, x_sub, edge_index_us, edge_index_su,
           W1_us, b1_us, W1_su, b1_su, W2_us, b2_us, W2_su, b2_su):
    sp_us = _pad_src(edge_index_us[0])
    dp_us = _pad_dst(edge_index_us[1])
    sp_su = _pad_src(edge_index_su[0])
    dp_su = _pad_dst(edge_index_su[1])

    sp12 = jnp.concatenate([sp_us, sp_su])
    dp12 = jnp.concatenate([dp_us, dp_su])
    degs = _degrees(sp12, dp12).reshape(4, ACC_H)
    degT = degs[:, :NU].T

    tbl_us, tbl_su, nrmT = _tc1(x_user, x_sub, W1_us, W1_su, degT)
    ns_us, nd_us = nrmT[:, 0], nrmT[:, 1]
    ns_su, nd_su = nrmT[:, 2], nrmT[:, 3]

    def _n2(a, b):
        return jnp.stack([a, b]).reshape(2, 1, NU_PAD)

    h_sub_s = _message_pass(
        tbl_us, jnp.concatenate([sp_us, sp_us]), jnp.concatenate([dp_us, dp_us]),
        _n2(nd_us, nd_us), _n2(ns_su, ns_su),
        b1_us.reshape(2, 1, 128), relu_ns=True)
    h_user_s = _message_pass(
        tbl_su, jnp.concatenate([sp_su, sp_su]), jnp.concatenate([dp_su, dp_su]),
        _n2(nd_su, nd_su), _n2(ns_us, ns_us),
        b1_su.reshape(2, 1, 128), relu_ns=True)

    tbl2, = _tc2(h_user_s, h_sub_s, W2_us, W2_su)

    outs = _message_pass(
        tbl2, sp12, dp12,
        _n2(nd_us, nd_su), _n2(nd_us, nd_su),
        jnp.stack([b2_us, b2_su]).reshape(2, 1, 128), relu_ns=False)

    return outs[1, :NU], outs[0, :NU]


# speedup vs baseline: 5.5740x; 1.0105x over previous
import functools

import jax
import jax.numpy as jnp
from jax import lax
from jax.experimental import pallas as pl
from jax.experimental.pallas import tpu as pltpu
from jax.experimental.pallas import tpu_sc as plsc

NU = 10000
E = 160000
D_IN = 256
D_H = 256
D_OUT = 128

NTILES = 16
CHUNK = 128
NCHUNK = 80
EPT = NCHUNK * CHUNK
E_PAD = NTILES * EPT
NPAD = E_PAD - E
TBL_N = 12288
NTRASH = 240
NU_PAD = 10240
ACC_H = 14336
ZCH_H = ACC_H // NTILES
RPT = NU_PAD // NTILES
RSTG = 64
IDXB = 16

_f32 = jnp.float32
_i32 = jnp.int32


def _pad_src(idx):
    pad = NU + jnp.arange(NPAD, dtype=_i32) % (TBL_N - NU)
    return jnp.concatenate([idx.astype(_i32), pad]).reshape(NTILES, NCHUNK, CHUNK)


def _pad_dst(idx):
    pad = NU + jnp.arange(NPAD, dtype=_i32) % NTRASH
    return jnp.concatenate([idx.astype(_i32), pad]).reshape(NTILES, NCHUNK, CHUNK)


def _deg_body(sp, dp, out, acc_s, acc_d, idx_s, idx_d, ones_v, zbuf):
    c = lax.axis_index("c")
    t = lax.axis_index("s")

    def _zi(i, carry):
        zbuf[pl.ds(i * 16, 16)] = jnp.zeros((16,), _f32)
        return carry
    lax.fori_loop(0, ZCH_H // 16, _zi, 0)

    def _oi(i, carry):
        ones_v[pl.ds(i * 16, 16)] = jnp.ones((16,), _f32)
        return carry
    lax.fori_loop(0, CHUNK // 16, _oi, 0)

    zoff = pl.multiple_of(t * ZCH_H, ZCH_H)
    pltpu.sync_copy(zbuf, acc_s.at[pl.ds(zoff, ZCH_H)])
    pltpu.sync_copy(zbuf, acc_d.at[pl.ds(zoff, ZCH_H)])
    w = c * NTILES + t
    pltpu.sync_copy(sp.at[w], idx_s)
    pltpu.sync_copy(dp.at[w], idx_d)
    plsc.subcore_barrier()

    def _ch(j, carry):
        pltpu.sync_copy(ones_v, acc_s.at[idx_s.at[j]], add=True)
        pltpu.sync_copy(ones_v, acc_d.at[idx_d.at[j]], add=True)
        return carry
    lax.fori_loop(0, NCHUNK, _ch, 0)
    plsc.subcore_barrier()

    pltpu.sync_copy(acc_s.at[pl.ds(zoff, ZCH_H)], out.at[2 * c, 0, pl.ds(zoff, ZCH_H)])
    pltpu.sync_copy(acc_d.at[pl.ds(zoff, ZCH_H)], out.at[2 * c + 1, 0, pl.ds(zoff, ZCH_H)])


def _degrees(sp12, dp12):
    mesh = plsc.VectorSubcoreMesh(core_axis_name="c", subcore_axis_name="s")
    return pl.kernel(
        _deg_body,
        out_type=jax.ShapeDtypeStruct((4, 1, ACC_H), _f32),
        mesh=mesh,
        scratch_types=[
            pltpu.VMEM_SHARED((ACC_H,), _f32),
            pltpu.VMEM_SHARED((ACC_H,), _f32),
            pltpu.VMEM((NCHUNK, CHUNK), _i32),
            pltpu.VMEM((NCHUNK, CHUNK), _i32),
            pltpu.VMEM((CHUNK,), _f32),
            pltpu.VMEM((ZCH_H,), _f32),
        ],
    )(sp12, dp12)


def _msg_body(relu_ns, nph, tbl, sp, dp, nd, ns, bias, out,
              acc, rows_a, rows_b, idx_s, idx_d, zbuf, stage, ndv, nsv, bv,
              sem_ga, sem_gb, sem_sa, sem_sb):
    c = lax.axis_index("c")
    t = lax.axis_index("s")
    tstart = pl.multiple_of(t * RPT, RPT)

    def _zi(i, carry):
        for g in range(8):
            zbuf[i, pl.ds(g * 16, 16)] = jnp.zeros((16,), _f32)
        return carry
    lax.fori_loop(0, 16, _zi, 0)

    bufs = (rows_a, rows_b)
    gsem = (sem_ga, sem_gb)
    ssem = (sem_sa, sem_sb)

    for phase in range(nph):
        sel = 2 * phase + c
        w = 32 * phase + c * NTILES + t

        def _za(k, carry):
            off = pl.multiple_of(t * RPT + k * 16, 16)
            pltpu.sync_copy(zbuf, acc.at[pl.ds(off, 16)])
            return carry
        lax.fori_loop(0, RPT // 16, _za, 0)

        pltpu.sync_copy(nd.at[sel, 0, pl.ds(tstart, RPT)], ndv.at[pl.ds(0, RPT)])
        if relu_ns:
            pltpu.sync_copy(ns.at[sel, 0, pl.ds(tstart, RPT)],
                            nsv.at[pl.ds(0, RPT)])
        pltpu.sync_copy(bias.at[sel, 0], bv)
        plsc.subcore_barrier()

        def _win(jw, carry):
            woff = pl.multiple_of(jw * IDXB, IDXB)
            pltpu.sync_copy(sp.at[w, pl.ds(woff, IDXB)], idx_s)
            pltpu.sync_copy(dp.at[w, pl.ds(woff, IDXB)], idx_d)

            gd = [None, None]
            sd = [None, None]
            for k in range(2):
                gd[k] = pltpu.async_copy(tbl.at[sel].at[idx_s.at[k]],
                                         bufs[k], gsem[k])
            for k in range(IDXB):
                b = k % 2
                gd[b].wait()
                sd[b] = pltpu.async_copy(bufs[b], acc.at[idx_d.at[k]], ssem[b],
                                         add=True)
                if k + 2 < IDXB:
                    sd[b].wait()
                    gd[b] = pltpu.async_copy(tbl.at[sel].at[idx_s.at[k + 2]],
                                             bufs[b], gsem[b])
            sd[0].wait()
            sd[1].wait()
            return carry
        lax.fori_loop(0, NCHUNK // IDXB, _win, 0)
        plsc.subcore_barrier()

        def _stage(st, carry):
            lo = pl.multiple_of(tstart + st * RSTG, RSTG)
            pltpu.sync_copy(acc.at[pl.ds(lo, RSTG)], stage)

            def _row(r, carry2):
                ndr = ndv[pl.ds(st * RSTG + r, 16)][0]
                nsr = nsv[pl.ds(st * RSTG + r, 16)][0] if relu_ns else None
                for g in range(8):
                    sl = pl.ds(g * 16, 16)
                    v = stage[r, sl] * ndr + bv[sl]
                    if relu_ns:
                        v = jnp.maximum(v, 0.0) * nsr
                    stage[r, sl] = v
                return carry2
            lax.fori_loop(0, RSTG, _row, 0)
            pltpu.sync_copy(stage, out.at[sel, pl.ds(lo, RSTG)])
            return carry
        lax.fori_loop(0, RPT // RSTG, _stage, 0)


def _message_pass(tbl, sp, dp, nd, ns, bias, relu_ns, nph):
    mesh = plsc.VectorSubcoreMesh(core_axis_name="c", subcore_axis_name="s")
    return pl.kernel(
        functools.partial(_msg_body, relu_ns, nph),
        out_type=jax.ShapeDtypeStruct((2 * nph, NU_PAD, 128), _f32),
        mesh=mesh,
        scratch_types=[
            pltpu.VMEM_SHARED((NU_PAD, 128), _f32),
            pltpu.VMEM((CHUNK, 128), _f32),
            pltpu.VMEM((CHUNK, 128), _f32),
            pltpu.VMEM((IDXB, CHUNK), _i32),
            pltpu.VMEM((IDXB, CHUNK), _i32),
            pltpu.VMEM((16, 128), _f32),
            pltpu.VMEM((RSTG, 128), _f32),
            pltpu.VMEM((RPT + 16,), _f32),
            pltpu.VMEM((RPT + 16,), _f32),
            pltpu.VMEM((128,), _f32),
            pltpu.SemaphoreType.DMA,
            pltpu.SemaphoreType.DMA,
            pltpu.SemaphoreType.DMA,
            pltpu.SemaphoreType.DMA,
        ],
    )(tbl, sp, dp, nd, ns, bias)


_RB = 2048


def _tc1a_body(xu, xs, wu, ws, mu, ms):
    mu[...] = jnp.dot(xu[...], wu[...], preferred_element_type=_f32)
    ms[...] = jnp.dot(xs[...], ws[...], preferred_element_type=_f32)


def _tc1a(x_user, x_sub, w1u, w1s):
    return pl.pallas_call(
        _tc1a_body,
        grid=(NU_PAD // _RB,),
        in_specs=[
            pl.BlockSpec((_RB, D_IN), lambda i: (i, 0)),
            pl.BlockSpec((_RB, D_IN), lambda i: (i, 0)),
            pl.BlockSpec((D_IN, D_H), lambda i: (0, 0)),
            pl.BlockSpec((D_IN, D_H), lambda i: (0, 0)),
        ],
        out_specs=[
            pl.BlockSpec((_RB, D_H), lambda i: (i, 0)),
            pl.BlockSpec((_RB, D_H), lambda i: (i, 0)),
        ],
        out_shape=[
            jax.ShapeDtypeStruct((NU_PAD, D_H), _f32),
            jax.ShapeDtypeStruct((NU_PAD, D_H), _f32),
        ],
    )(x_user, x_sub, w1u, w1s)


def _tc1b_body(mu, ms, dg, tb, nrm):
    nb = lax.rsqrt(jnp.maximum(dg[...], 1.0))
    nrm[...] = nb
    hu = mu[...] * nb[:, 0:1]
    hs = ms[...] * nb[:, 2:3]
    tb[...] = jnp.stack([hu[:, 0:128], hu[:, 128:256],
                         hs[:, 0:128], hs[:, 128:256]])


def _tc1b(mu, ms, degT):
    return pl.pallas_call(
        _tc1b_body,
        grid=(NU_PAD // _RB,),
        in_specs=[
            pl.BlockSpec((_RB, D_H), lambda i: (i, 0)),
            pl.BlockSpec((_RB, D_H), lambda i: (i, 0)),
            pl.BlockSpec((_RB, 4), lambda i: (i, 0)),
        ],
        out_specs=[
            pl.BlockSpec((4, _RB, 128), lambda i: (0, i, 0)),
            pl.BlockSpec((_RB, 4), lambda i: (i, 0)),
        ],
        out_shape=[
            jax.ShapeDtypeStruct((4, TBL_N, 128), _f32),
            jax.ShapeDtypeStruct((NU_PAD, 4), _f32),
        ],
    )(mu, ms, degT)


def _tc2_body(h4, wu, ws, tb2):
    gu = (jnp.dot(h4[2], wu[0:128, :], preferred_element_type=_f32)
          + jnp.dot(h4[3], wu[128:256, :], preferred_element_type=_f32))
    gs = (jnp.dot(h4[0], ws[0:128, :], preferred_element_type=_f32)
          + jnp.dot(h4[1], ws[128:256, :], preferred_element_type=_f32))
    tb2[...] = jnp.stack([gu, gs])


def _tc2(h4, w2u, w2s):
    return pl.pallas_call(
        _tc2_body,
        grid=(NU_PAD // _RB,),
        in_specs=[
            pl.BlockSpec((4, _RB, 128), lambda i: (0, i, 0)),
            pl.BlockSpec((D_H, D_OUT), lambda i: (0, 0)),
            pl.BlockSpec((D_H, D_OUT), lambda i: (0, 0)),
        ],
        out_specs=[
            pl.BlockSpec((2, _RB, 128), lambda i: (0, i, 0)),
        ],
        out_shape=[
            jax.ShapeDtypeStruct((2, TBL_N, 128), _f32),
        ],
    )(h4, w2u, w2s)


def kernel(x_user, x_sub, edge_index_us, edge_index_su,
           W1_us, b1_us, W1_su, b1_su, W2_us, b2_us, W2_su, b2_su):
    sp_us = _pad_src(edge_index_us[0])
    dp_us = _pad_dst(edge_index_us[1])
    sp_su = _pad_src(edge_index_su[0])
    dp_su = _pad_dst(edge_index_su[1])

    sp12 = jnp.concatenate([sp_us, sp_su])
    dp12 = jnp.concatenate([dp_us, dp_su])
    mu, ms = _tc1a(x_user, x_sub, W1_us, W1_su)
    degs = _degrees(sp12, dp12).reshape(4, ACC_H)
    degT = degs[:, :NU].T

    tbl1, nrmT = _tc1b(mu, ms, degT)
    ns_us, nd_us = nrmT[:, 0], nrmT[:, 1]
    ns_su, nd_su = nrmT[:, 2], nrmT[:, 3]

    h4 = _message_pass(
        tbl1,
        jnp.concatenate([sp_us, sp_us, sp_su, sp_su]),
        jnp.concatenate([dp_us, dp_us, dp_su, dp_su]),
        jnp.stack([nd_us, nd_us, nd_su, nd_su]).reshape(4, 1, NU_PAD),
        jnp.stack([ns_su, ns_su, ns_us, ns_us]).reshape(4, 1, NU_PAD),
        jnp.concatenate([b1_us, b1_su]).reshape(4, 1, 128),
        relu_ns=True, nph=2)

    tbl2, = _tc2(h4, W2_us, W2_su)

    outs = _message_pass(
        tbl2, sp12, dp12,
        jnp.stack([nd_us, nd_su]).reshape(2, 1, NU_PAD),
        jnp.stack([nd_us, nd_su]).reshape(2, 1, NU_PAD),
        jnp.stack([b2_us, b2_su]).reshape(2, 1, 128),
        relu_ns=False, nph=1)

    return outs[1, :NU], outs[0, :NU]


# speedup vs baseline: 6.8652x; 1.2316x over previous
import functools

import jax
import jax.numpy as jnp
from jax import lax
from jax.experimental import pallas as pl
from jax.experimental.pallas import tpu as pltpu
from jax.experimental.pallas import tpu_sc as plsc

NU = 10000
E = 160000
D_IN = 256
D_H = 256
D_OUT = 128

NTILES = 16
CHUNK = 128
NCHUNK = 80
EPT = NCHUNK * CHUNK
E_PAD = NTILES * EPT
NPAD = E_PAD - E
TBL_N = 12288
NTRASH = 240
NU_PAD = 10240
ACC_H = 14336
ZCH_H = ACC_H // NTILES
RPT = NU_PAD // NTILES
RSTG = 64
IDXB = 16

_f32 = jnp.float32
_i32 = jnp.int32


def _pad_src(idx):
    pad = NU + jnp.arange(NPAD, dtype=_i32) % (TBL_N - NU)
    return jnp.concatenate([idx.astype(_i32), pad]).reshape(NTILES, NCHUNK, CHUNK)


def _pad_dst(idx):
    pad = NU + jnp.arange(NPAD, dtype=_i32) % NTRASH
    return jnp.concatenate([idx.astype(_i32), pad]).reshape(NTILES, NCHUNK, CHUNK)


def _deg_body(sp, dp, out, acc_s, acc_d, idx_s, idx_d, ones_v, zbuf):
    c = lax.axis_index("c")
    t = lax.axis_index("s")

    def _zi(i, carry):
        zbuf[pl.ds(i * 16, 16)] = jnp.zeros((16,), _f32)
        return carry
    lax.fori_loop(0, ZCH_H // 16, _zi, 0)

    def _oi(i, carry):
        ones_v[pl.ds(i * 16, 16)] = jnp.ones((16,), _f32)
        return carry
    lax.fori_loop(0, CHUNK // 16, _oi, 0)

    zoff = pl.multiple_of(t * ZCH_H, ZCH_H)
    pltpu.sync_copy(zbuf, acc_s.at[pl.ds(zoff, ZCH_H)])
    pltpu.sync_copy(zbuf, acc_d.at[pl.ds(zoff, ZCH_H)])
    w = c * NTILES + t
    pltpu.sync_copy(sp.at[w], idx_s)
    pltpu.sync_copy(dp.at[w], idx_d)
    plsc.subcore_barrier()

    def _ch(j, carry):
        pltpu.sync_copy(ones_v, acc_s.at[idx_s.at[j]], add=True)
        pltpu.sync_copy(ones_v, acc_d.at[idx_d.at[j]], add=True)
        return carry
    lax.fori_loop(0, NCHUNK, _ch, 0)
    plsc.subcore_barrier()

    pltpu.sync_copy(acc_s.at[pl.ds(zoff, ZCH_H)], out.at[2 * c, 0, pl.ds(zoff, ZCH_H)])
    pltpu.sync_copy(acc_d.at[pl.ds(zoff, ZCH_H)], out.at[2 * c + 1, 0, pl.ds(zoff, ZCH_H)])


def _degrees(sp12, dp12):
    mesh = plsc.VectorSubcoreMesh(core_axis_name="c", subcore_axis_name="s")
    return pl.kernel(
        _deg_body,
        out_type=jax.ShapeDtypeStruct((4, 1, ACC_H), _f32),
        mesh=mesh,
        scratch_types=[
            pltpu.VMEM_SHARED((ACC_H,), _f32),
            pltpu.VMEM_SHARED((ACC_H,), _f32),
            pltpu.VMEM((NCHUNK, CHUNK), _i32),
            pltpu.VMEM((NCHUNK, CHUNK), _i32),
            pltpu.VMEM((CHUNK,), _f32),
            pltpu.VMEM((ZCH_H,), _f32),
        ],
    )(sp12, dp12)


def _msg_body(nph, tbl, sp, dp, out,
              acc, rows_a, rows_b, idx_s, idx_d, zbuf,
              sem_ga, sem_gb, sem_sa, sem_sb, sem_z):
    c = lax.axis_index("c")
    t = lax.axis_index("s")
    tstart = pl.multiple_of(t * RPT, RPT)

    def _zi(i, carry):
        for g in range(8):
            zbuf[i, pl.ds(g * 16, 16)] = jnp.zeros((16,), _f32)
        return carry
    lax.fori_loop(0, 16, _zi, 0)

    bufs = (rows_a, rows_b)
    gsem = (sem_ga, sem_gb)
    ssem = (sem_sa, sem_sb)

    for phase in range(nph):
        sel = 2 * phase + c
        w = 32 * phase + c * NTILES + t

        zd = []
        for k in range(RPT // 16):
            off = pl.multiple_of(t * RPT + k * 16, 16)
            zd.append(pltpu.async_copy(zbuf, acc.at[pl.ds(off, 16)], sem_z))
        for d in zd:
            d.wait()
        plsc.subcore_barrier()

        def _win(jw, carry):
            woff = pl.multiple_of(jw * IDXB, IDXB)
            pltpu.sync_copy(sp.at[w, pl.ds(woff, IDXB)], idx_s)
            pltpu.sync_copy(dp.at[w, pl.ds(woff, IDXB)], idx_d)

            gd = [None, None]
            sd = [None, None]
            for k in range(2):
                gd[k] = pltpu.async_copy(tbl.at[sel].at[idx_s.at[k]],
                                         bufs[k], gsem[k])
            for k in range(IDXB):
                b = k % 2
                gd[b].wait()
                sd[b] = pltpu.async_copy(bufs[b], acc.at[idx_d.at[k]], ssem[b],
                                         add=True)
                if k + 2 < IDXB:
                    sd[b].wait()
                    gd[b] = pltpu.async_copy(tbl.at[sel].at[idx_s.at[k + 2]],
                                             bufs[b], gsem[b])
            sd[0].wait()
            sd[1].wait()
            return carry
        lax.fori_loop(0, NCHUNK // IDXB, _win, 0)
        plsc.subcore_barrier()

        pltpu.sync_copy(acc.at[pl.ds(tstart, RPT)],
                        out.at[sel, pl.ds(tstart, RPT)])


def _message_pass(tbl, sp, dp, nph):
    mesh = plsc.VectorSubcoreMesh(core_axis_name="c", subcore_axis_name="s")
    return pl.kernel(
        functools.partial(_msg_body, nph),
        out_type=jax.ShapeDtypeStruct((2 * nph, NU_PAD, 128), _f32),
        mesh=mesh,
        scratch_types=[
            pltpu.VMEM_SHARED((NU_PAD, 128), _f32),
            pltpu.VMEM((CHUNK, 128), _f32),
            pltpu.VMEM((CHUNK, 128), _f32),
            pltpu.VMEM((IDXB, CHUNK), _i32),
            pltpu.VMEM((IDXB, CHUNK), _i32),
            pltpu.VMEM((16, 128), _f32),
            pltpu.SemaphoreType.DMA,
            pltpu.SemaphoreType.DMA,
            pltpu.SemaphoreType.DMA,
            pltpu.SemaphoreType.DMA,
            pltpu.SemaphoreType.DMA,
        ],
    )(tbl, sp, dp)


_RB = 2048


def _tc1a_body(xu, xs, wu, ws, mu, ms):
    mu[...] = jnp.dot(xu[...], wu[...], preferred_element_type=_f32)
    ms[...] = jnp.dot(xs[...], ws[...], preferred_element_type=_f32)


def _tc1a(x_user, x_sub, w1u, w1s):
    return pl.pallas_call(
        _tc1a_body,
        grid=(NU_PAD // _RB,),
        in_specs=[
            pl.BlockSpec((_RB, D_IN), lambda i: (i, 0)),
            pl.BlockSpec((_RB, D_IN), lambda i: (i, 0)),
            pl.BlockSpec((D_IN, D_H), lambda i: (0, 0)),
            pl.BlockSpec((D_IN, D_H), lambda i: (0, 0)),
        ],
        out_specs=[
            pl.BlockSpec((_RB, D_H), lambda i: (i, 0)),
            pl.BlockSpec((_RB, D_H), lambda i: (i, 0)),
        ],
        out_shape=[
            jax.ShapeDtypeStruct((NU_PAD, D_H), _f32),
            jax.ShapeDtypeStruct((NU_PAD, D_H), _f32),
        ],
    )(x_user, x_sub, w1u, w1s)


def _tc1b_body(mu, ms, dg, tb, nrm):
    nb = lax.rsqrt(jnp.maximum(dg[...], 1.0))
    nrm[...] = nb
    hu = mu[...] * nb[:, 0:1]
    hs = ms[...] * nb[:, 2:3]
    tb[...] = jnp.stack([hu[:, 0:128], hu[:, 128:256],
                         hs[:, 0:128], hs[:, 128:256]])


def _tc1b(mu, ms, degT):
    return pl.pallas_call(
        _tc1b_body,
        grid=(NU_PAD // _RB,),
        in_specs=[
            pl.BlockSpec((_RB, D_H), lambda i: (i, 0)),
            pl.BlockSpec((_RB, D_H), lambda i: (i, 0)),
            pl.BlockSpec((_RB, 4), lambda i: (i, 0)),
        ],
        out_specs=[
            pl.BlockSpec((4, _RB, 128), lambda i: (0, i, 0)),
            pl.BlockSpec((_RB, 4), lambda i: (i, 0)),
        ],
        out_shape=[
            jax.ShapeDtypeStruct((4, TBL_N, 128), _f32),
            jax.ShapeDtypeStruct((NU_PAD, 4), _f32),
        ],
    )(mu, ms, degT)


def _tc2_body(m4, nrm, b1, wu, ws, tb2):
    nb = nrm[...]
    b = b1[...]
    h0 = jnp.maximum(m4[0] * nb[:, 1:2] + b[0:1, :], 0.0) * nb[:, 2:3]
    h1 = jnp.maximum(m4[1] * nb[:, 1:2] + b[1:2, :], 0.0) * nb[:, 2:3]
    h2 = jnp.maximum(m4[2] * nb[:, 3:4] + b[2:3, :], 0.0) * nb[:, 0:1]
    h3 = jnp.maximum(m4[3] * nb[:, 3:4] + b[3:4, :], 0.0) * nb[:, 0:1]
    gu = (jnp.dot(h2, wu[0:128, :], preferred_element_type=_f32)
          + jnp.dot(h3, wu[128:256, :], preferred_element_type=_f32))
    gs = (jnp.dot(h0, ws[0:128, :], preferred_element_type=_f32)
          + jnp.dot(h1, ws[128:256, :], preferred_element_type=_f32))
    tb2[...] = jnp.stack([gu, gs])


def _tc2(m4, nrmT, b1, w2u, w2s):
    return pl.pallas_call(
        _tc2_body,
        grid=(NU_PAD // _RB,),
        in_specs=[
            pl.BlockSpec((4, _RB, 128), lambda i: (0, i, 0)),
            pl.BlockSpec((_RB, 4), lambda i: (i, 0)),
            pl.BlockSpec((4, 128), lambda i: (0, 0)),
            pl.BlockSpec((D_H, D_OUT), lambda i: (0, 0)),
            pl.BlockSpec((D_H, D_OUT), lambda i: (0, 0)),
        ],
        out_specs=[
            pl.BlockSpec((2, _RB, 128), lambda i: (0, i, 0)),
        ],
        out_shape=[
            jax.ShapeDtypeStruct((2, TBL_N, 128), _f32),
        ],
    )(m4, nrmT, b1, w2u, w2s)


def _tc3_body(m2, nrm, b2, ou, osub):
    nb = nrm[...]
    osub[...] = m2[0] * nb[:, 1:2] + b2[0:1, :]
    ou[...] = m2[1] * nb[:, 3:4] + b2[1:2, :]


def _tc3(m2, nrmT, b2):
    return pl.pallas_call(
        _tc3_body,
        grid=(NU_PAD // _RB,),
        in_specs=[
            pl.BlockSpec((2, _RB, 128), lambda i: (0, i, 0)),
            pl.BlockSpec((_RB, 4), lambda i: (i, 0)),
            pl.BlockSpec((2, 128), lambda i: (0, 0)),
        ],
        out_specs=[
            pl.BlockSpec((_RB, 128), lambda i: (i, 0)),
            pl.BlockSpec((_RB, 128), lambda i: (i, 0)),
        ],
        out_shape=[
            jax.ShapeDtypeStruct((NU_PAD, 128), _f32),
            jax.ShapeDtypeStruct((NU_PAD, 128), _f32),
        ],
    )(m2, nrmT, b2)


def kernel(x_user, x_sub, edge_index_us, edge_index_su,
           W1_us, b1_us, W1_su, b1_su, W2_us, b2_us, W2_su, b2_su):
    sp_us = _pad_src(edge_index_us[0])
    dp_us = _pad_dst(edge_index_us[1])
    sp_su = _pad_src(edge_index_su[0])
    dp_su = _pad_dst(edge_index_su[1])

    sp12 = jnp.concatenate([sp_us, sp_su])
    dp12 = jnp.concatenate([dp_us, dp_su])
    mu, ms = _tc1a(x_user, x_sub, W1_us, W1_su)
    degs = _degrees(sp12, dp12).reshape(4, ACC_H)
    degT = degs[:, :NU].T

    tbl1, nrmT = _tc1b(mu, ms, degT)

    m4 = _message_pass(
        tbl1,
        jnp.concatenate([sp_us, sp_us, sp_su, sp_su]),
        jnp.concatenate([dp_us, dp_us, dp_su, dp_su]),
        nph=2)

    tbl2, = _tc2(m4, nrmT, jnp.concatenate([b1_us, b1_su]).reshape(4, 128),
                 W2_us, W2_su)

    m2 = _message_pass(tbl2, sp12, dp12, nph=1)

    out_user, out_sub = _tc3(m2, nrmT, jnp.stack([b2_us, b2_su]))
    return out_user[:NU], out_sub[:NU]


# speedup vs baseline: 7.0931x; 1.0332x over previous
import functools

import jax
import jax.numpy as jnp
from jax import lax
from jax.experimental import pallas as pl
from jax.experimental.pallas import tpu as pltpu
from jax.experimental.pallas import tpu_sc as plsc

NU = 10000
E = 160000
D_IN = 256
D_H = 256
D_OUT = 128

NTILES = 16
CHUNK = 128
NCHUNK = 80
EPT = NCHUNK * CHUNK
E_PAD = NTILES * EPT
NPAD = E_PAD - E
TBL_N = 12288
NTRASH = 240
NU_PAD = 10240
ACC_H = 14336
ZCH_H = ACC_H // NTILES
RPT = NU_PAD // NTILES
RSTG = 64
IDXB = 16
MWIN = 40

_f32 = jnp.float32
_i32 = jnp.int32


def _pad_src(idx):
    pad = NU + jnp.arange(NPAD, dtype=_i32) % (TBL_N - NU)
    return jnp.concatenate([idx.astype(_i32), pad]).reshape(NTILES, NCHUNK, CHUNK)


def _pad_dst(idx):
    pad = NU + jnp.arange(NPAD, dtype=_i32) % NTRASH
    return jnp.concatenate([idx.astype(_i32), pad]).reshape(NTILES, NCHUNK, CHUNK)


def _deg_body(sp, dp, out, acc_s, acc_d, idx_s, idx_d, ones_v, zbuf):
    c = lax.axis_index("c")
    t = lax.axis_index("s")

    def _zi(i, carry):
        zbuf[pl.ds(i * 16, 16)] = jnp.zeros((16,), _f32)
        return carry
    lax.fori_loop(0, ZCH_H // 16, _zi, 0)

    def _oi(i, carry):
        ones_v[pl.ds(i * 16, 16)] = jnp.ones((16,), _f32)
        return carry
    lax.fori_loop(0, CHUNK // 16, _oi, 0)

    zoff = pl.multiple_of(t * ZCH_H, ZCH_H)
    pltpu.sync_copy(zbuf, acc_s.at[pl.ds(zoff, ZCH_H)])
    pltpu.sync_copy(zbuf, acc_d.at[pl.ds(zoff, ZCH_H)])
    w = c * NTILES + t
    pltpu.sync_copy(sp.at[w], idx_s)
    pltpu.sync_copy(dp.at[w], idx_d)
    plsc.subcore_barrier()

    def _ch(j, carry):
        pltpu.sync_copy(ones_v, acc_s.at[idx_s.at[j]], add=True)
        pltpu.sync_copy(ones_v, acc_d.at[idx_d.at[j]], add=True)
        return carry
    lax.fori_loop(0, NCHUNK, _ch, 0)
    plsc.subcore_barrier()

    pltpu.sync_copy(acc_s.at[pl.ds(zoff, ZCH_H)], out.at[2 * c, 0, pl.ds(zoff, ZCH_H)])
    pltpu.sync_copy(acc_d.at[pl.ds(zoff, ZCH_H)], out.at[2 * c + 1, 0, pl.ds(zoff, ZCH_H)])


def _degrees(sp12, dp12):
    mesh = plsc.VectorSubcoreMesh(core_axis_name="c", subcore_axis_name="s")
    return pl.kernel(
        _deg_body,
        out_type=jax.ShapeDtypeStruct((4, 1, ACC_H), _f32),
        mesh=mesh,
        scratch_types=[
            pltpu.VMEM_SHARED((ACC_H,), _f32),
            pltpu.VMEM_SHARED((ACC_H,), _f32),
            pltpu.VMEM((NCHUNK, CHUNK), _i32),
            pltpu.VMEM((NCHUNK, CHUNK), _i32),
            pltpu.VMEM((CHUNK,), _f32),
            pltpu.VMEM((ZCH_H,), _f32),
        ],
    )(sp12, dp12)


def _msg_body(nph, tbl, sp, dp, out,
              acc, rows_a, rows_b, idx_s, idx_d, zbuf,
              sem_ga, sem_gb, sem_sa, sem_sb, sem_z):
    c = lax.axis_index("c")
    t = lax.axis_index("s")
    tstart = pl.multiple_of(t * RPT, RPT)

    def _zi(i, carry):
        for g in range(8):
            zbuf[i, pl.ds(g * 16, 16)] = jnp.zeros((16,), _f32)
        return carry
    lax.fori_loop(0, 16, _zi, 0)

    bufs = (rows_a, rows_b)
    gsem = (sem_ga, sem_gb)
    ssem = (sem_sa, sem_sb)

    for phase in range(nph):
        sel = 2 * phase + c
        w = 32 * phase + c * NTILES + t

        zd = []
        for k in range(RPT // 16):
            off = pl.multiple_of(t * RPT + k * 16, 16)
            zd.append(pltpu.async_copy(zbuf, acc.at[pl.ds(off, 16)], sem_z))
        for d in zd:
            d.wait()
        plsc.subcore_barrier()

        for mw in range(NCHUNK // MWIN):
            moff = MWIN * mw
            pltpu.sync_copy(sp.at[w, pl.ds(moff, MWIN)], idx_s)
            pltpu.sync_copy(dp.at[w, pl.ds(moff, MWIN)], idx_d)
            for b in range(2):
                pltpu.async_copy(tbl.at[sel].at[idx_s.at[b]], bufs[b], gsem[b])

            def _pair(j, carry):
                for b in range(2):
                    k = 2 * j + b
                    pltpu.make_async_copy(tbl.at[sel].at[idx_s.at[k]],
                                          bufs[b], gsem[b]).wait()
                    sd = pltpu.async_copy(bufs[b], acc.at[idx_d.at[k]],
                                          ssem[b], add=True)
                    sd.wait()
                    kn = jnp.minimum(k + 2, MWIN - 1)
                    pltpu.async_copy(tbl.at[sel].at[idx_s.at[kn]],
                                     bufs[b], gsem[b])
                return carry
            lax.fori_loop(0, MWIN // 2, _pair, 0)
            for b in range(2):
                pltpu.make_async_copy(tbl.at[sel].at[idx_s.at[0]],
                                      bufs[b], gsem[b]).wait()
        plsc.subcore_barrier()

        pltpu.sync_copy(acc.at[pl.ds(tstart, RPT)],
                        out.at[sel, pl.ds(tstart, RPT)])


def _message_pass(tbl, sp, dp, nph):
    mesh = plsc.VectorSubcoreMesh(core_axis_name="c", subcore_axis_name="s")
    return pl.kernel(
        functools.partial(_msg_body, nph),
        out_type=jax.ShapeDtypeStruct((2 * nph, NU_PAD, 128), _f32),
        mesh=mesh,
        scratch_types=[
            pltpu.VMEM_SHARED((NU_PAD, 128), _f32),
            pltpu.VMEM((CHUNK, 128), _f32),
            pltpu.VMEM((CHUNK, 128), _f32),
            pltpu.VMEM((MWIN, CHUNK), _i32),
            pltpu.VMEM((MWIN, CHUNK), _i32),
            pltpu.VMEM((16, 128), _f32),
            pltpu.SemaphoreType.DMA,
            pltpu.SemaphoreType.DMA,
            pltpu.SemaphoreType.DMA,
            pltpu.SemaphoreType.DMA,
            pltpu.SemaphoreType.DMA,
        ],
    )(tbl, sp, dp)


_RB = 2048


def _tc1a_body(xu, xs, wu, ws, mu, ms):
    mu[...] = jnp.dot(xu[...], wu[...], preferred_element_type=_f32)
    ms[...] = jnp.dot(xs[...], ws[...], preferred_element_type=_f32)


def _tc1a(x_user, x_sub, w1u, w1s):
    return pl.pallas_call(
        _tc1a_body,
        grid=(NU_PAD // _RB,),
        in_specs=[
            pl.BlockSpec((_RB, D_IN), lambda i: (i, 0)),
            pl.BlockSpec((_RB, D_IN), lambda i: (i, 0)),
            pl.BlockSpec((D_IN, D_H), lambda i: (0, 0)),
            pl.BlockSpec((D_IN, D_H), lambda i: (0, 0)),
        ],
        out_specs=[
            pl.BlockSpec((_RB, D_H), lambda i: (i, 0)),
            pl.BlockSpec((_RB, D_H), lambda i: (i, 0)),
        ],
        out_shape=[
            jax.ShapeDtypeStruct((NU_PAD, D_H), _f32),
            jax.ShapeDtypeStruct((NU_PAD, D_H), _f32),
        ],
    )(x_user, x_sub, w1u, w1s)


def _tc1b_body(mu, ms, dg, tb, nrm):
    nb = lax.rsqrt(jnp.maximum(dg[...], 1.0))
    nrm[...] = nb
    hu = mu[...] * nb[:, 0:1]
    hs = ms[...] * nb[:, 2:3]
    tb[...] = jnp.stack([hu[:, 0:128], hu[:, 128:256],
                         hs[:, 0:128], hs[:, 128:256]])


def _tc1b(mu, ms, degT):
    return pl.pallas_call(
        _tc1b_body,
        grid=(NU_PAD // _RB,),
        in_specs=[
            pl.BlockSpec((_RB, D_H), lambda i: (i, 0)),
            pl.BlockSpec((_RB, D_H), lambda i: (i, 0)),
            pl.BlockSpec((_RB, 4), lambda i: (i, 0)),
        ],
        out_specs=[
            pl.BlockSpec((4, _RB, 128), lambda i: (0, i, 0)),
            pl.BlockSpec((_RB, 4), lambda i: (i, 0)),
        ],
        out_shape=[
            jax.ShapeDtypeStruct((4, TBL_N, 128), _f32),
            jax.ShapeDtypeStruct((NU_PAD, 4), _f32),
        ],
    )(mu, ms, degT)


def _tc2_body(m4, nrm, b1, wu, ws, tb2):
    nb = nrm[...]
    b = b1[...]
    h0 = jnp.maximum(m4[0] * nb[:, 1:2] + b[0:1, :], 0.0) * nb[:, 2:3]
    h1 = jnp.maximum(m4[1] * nb[:, 1:2] + b[1:2, :], 0.0) * nb[:, 2:3]
    h2 = jnp.maximum(m4[2] * nb[:, 3:4] + b[2:3, :], 0.0) * nb[:, 0:1]
    h3 = jnp.maximum(m4[3] * nb[:, 3:4] + b[3:4, :], 0.0) * nb[:, 0:1]
    gu = (jnp.dot(h2, wu[0:128, :], preferred_element_type=_f32)
          + jnp.dot(h3, wu[128:256, :], preferred_element_type=_f32))
    gs = (jnp.dot(h0, ws[0:128, :], preferred_element_type=_f32)
          + jnp.dot(h1, ws[128:256, :], preferred_element_type=_f32))
    tb2[...] = jnp.stack([gu, gs])


def _tc2(m4, nrmT, b1, w2u, w2s):
    return pl.pallas_call(
        _tc2_body,
        grid=(NU_PAD // _RB,),
        in_specs=[
            pl.BlockSpec((4, _RB, 128), lambda i: (0, i, 0)),
            pl.BlockSpec((_RB, 4), lambda i: (i, 0)),
            pl.BlockSpec((4, 128), lambda i: (0, 0)),
            pl.BlockSpec((D_H, D_OUT), lambda i: (0, 0)),
            pl.BlockSpec((D_H, D_OUT), lambda i: (0, 0)),
        ],
        out_specs=[
            pl.BlockSpec((2, _RB, 128), lambda i: (0, i, 0)),
        ],
        out_shape=[
            jax.ShapeDtypeStruct((2, TBL_N, 128), _f32),
        ],
    )(m4, nrmT, b1, w2u, w2s)


def _tc3_body(m2, nrm, b2, ou, osub):
    nb = nrm[...]
    osub[...] = m2[0] * nb[:, 1:2] + b2[0:1, :]
    ou[...] = m2[1] * nb[:, 3:4] + b2[1:2, :]


def _tc3(m2, nrmT, b2):
    return pl.pallas_call(
        _tc3_body,
        grid=(NU_PAD // _RB,),
        in_specs=[
            pl.BlockSpec((2, _RB, 128), lambda i: (0, i, 0)),
            pl.BlockSpec((_RB, 4), lambda i: (i, 0)),
            pl.BlockSpec((2, 128), lambda i: (0, 0)),
        ],
        out_specs=[
            pl.BlockSpec((_RB, 128), lambda i: (i, 0)),
            pl.BlockSpec((_RB, 128), lambda i: (i, 0)),
        ],
        out_shape=[
            jax.ShapeDtypeStruct((NU_PAD, 128), _f32),
            jax.ShapeDtypeStruct((NU_PAD, 128), _f32),
        ],
    )(m2, nrmT, b2)


def kernel(x_user, x_sub, edge_index_us, edge_index_su,
           W1_us, b1_us, W1_su, b1_su, W2_us, b2_us, W2_su, b2_su):
    sp_us = _pad_src(edge_index_us[0])
    dp_us = _pad_dst(edge_index_us[1])
    sp_su = _pad_src(edge_index_su[0])
    dp_su = _pad_dst(edge_index_su[1])

    sp12 = jnp.concatenate([sp_us, sp_su])
    dp12 = jnp.concatenate([dp_us, dp_su])
    mu, ms = _tc1a(x_user, x_sub, W1_us, W1_su)
    degs = _degrees(sp12, dp12).reshape(4, ACC_H)
    degT = degs[:, :NU].T

    tbl1, nrmT = _tc1b(mu, ms, degT)

    m4 = _message_pass(
        tbl1,
        jnp.concatenate([sp_us, sp_us, sp_su, sp_su]),
        jnp.concatenate([dp_us, dp_us, dp_su, dp_su]),
        nph=2)

    tbl2, = _tc2(m4, nrmT, jnp.concatenate([b1_us, b1_su]).reshape(4, 128),
                 W2_us, W2_su)

    m2 = _message_pass(tbl2, sp12, dp12, nph=1)

    out_user, out_sub = _tc3(m2, nrmT, jnp.stack([b2_us, b2_su]))
    return out_user[:NU], out_sub[:NU]


# speedup vs baseline: 7.2811x; 1.0265x over previous
import functools

import jax
import jax.numpy as jnp
from jax import lax
from jax.experimental import pallas as pl
from jax.experimental.pallas import tpu as pltpu
from jax.experimental.pallas import tpu_sc as plsc

NU = 10000
E = 160000
D_IN = 256
D_H = 256
D_OUT = 128

NTILES = 16
CHUNK = 128
NCHUNK = 80
EPT = NCHUNK * CHUNK
E_PAD = NTILES * EPT
NPAD = E_PAD - E
TBL_N = 12288
NTRASH = 240
NU_PAD = 10240
ACC_H = 14336
ZCH_H = ACC_H // NTILES
RPT = NU_PAD // NTILES
RSTG = 64
IDXB = 16
MWIN = 40

_f32 = jnp.float32
_i32 = jnp.int32


def _pad_src(idx):
    pad = NU + jnp.arange(NPAD, dtype=_i32) % (TBL_N - NU)
    return jnp.concatenate([idx.astype(_i32), pad]).reshape(NTILES, NCHUNK, CHUNK)


def _pad_dst(idx):
    pad = NU + jnp.arange(NPAD, dtype=_i32) % NTRASH
    return jnp.concatenate([idx.astype(_i32), pad]).reshape(NTILES, NCHUNK, CHUNK)


def _deg_body(sp, dp, out, acc_s, acc_d, idx_s, idx_d, ones_v, zbuf, sem_h):
    c = lax.axis_index("c")
    t = lax.axis_index("s")

    def _zi(i, carry):
        zbuf[pl.ds(i * 16, 16)] = jnp.zeros((16,), _f32)
        return carry
    lax.fori_loop(0, ZCH_H // 16, _zi, 0)

    def _oi(i, carry):
        ones_v[pl.ds(i * 16, 16)] = jnp.ones((16,), _f32)
        return carry
    lax.fori_loop(0, CHUNK // 16, _oi, 0)

    zoff = pl.multiple_of(t * ZCH_H, ZCH_H)
    pltpu.sync_copy(zbuf, acc_s.at[pl.ds(zoff, ZCH_H)])
    pltpu.sync_copy(zbuf, acc_d.at[pl.ds(zoff, ZCH_H)])
    w = c * NTILES + t
    pltpu.sync_copy(sp.at[w], idx_s)
    pltpu.sync_copy(dp.at[w], idx_d)
    plsc.subcore_barrier()

    def _ch(j, carry):
        pltpu.async_copy(ones_v, acc_s.at[idx_s.at[j]], sem_h, add=True)
        pltpu.async_copy(ones_v, acc_d.at[idx_d.at[j]], sem_h, add=True)
        return carry
    lax.fori_loop(0, NCHUNK, _ch, 0)

    def _dr(j, carry):
        pltpu.make_async_copy(ones_v, acc_s.at[idx_s.at[0]], sem_h).wait()
        pltpu.make_async_copy(ones_v, acc_d.at[idx_d.at[0]], sem_h).wait()
        return carry
    lax.fori_loop(0, NCHUNK, _dr, 0)
    plsc.subcore_barrier()

    pltpu.sync_copy(acc_s.at[pl.ds(zoff, ZCH_H)], out.at[2 * c, 0, pl.ds(zoff, ZCH_H)])
    pltpu.sync_copy(acc_d.at[pl.ds(zoff, ZCH_H)], out.at[2 * c + 1, 0, pl.ds(zoff, ZCH_H)])


def _degrees(sp12, dp12):
    mesh = plsc.VectorSubcoreMesh(core_axis_name="c", subcore_axis_name="s")
    return pl.kernel(
        _deg_body,
        out_type=jax.ShapeDtypeStruct((4, 1, ACC_H), _f32),
        mesh=mesh,
        scratch_types=[
            pltpu.VMEM_SHARED((ACC_H,), _f32),
            pltpu.VMEM_SHARED((ACC_H,), _f32),
            pltpu.VMEM((NCHUNK, CHUNK), _i32),
            pltpu.VMEM((NCHUNK, CHUNK), _i32),
            pltpu.VMEM((CHUNK,), _f32),
            pltpu.VMEM((ZCH_H,), _f32),
            pltpu.SemaphoreType.DMA,
        ],
    )(sp12, dp12)


def _msg_body(nph, tbl, sp, dp, out,
              acc, rows_a, rows_b, idx_s, idx_d, zbuf,
              sem_ga, sem_gb, sem_sa, sem_sb, sem_z):
    c = lax.axis_index("c")
    t = lax.axis_index("s")
    tstart = pl.multiple_of(t * RPT, RPT)

    def _zi(i, carry):
        for g in range(8):
            zbuf[i, pl.ds(g * 16, 16)] = jnp.zeros((16,), _f32)
        return carry
    lax.fori_loop(0, 16, _zi, 0)

    bufs = (rows_a, rows_b)
    gsem = (sem_ga, sem_gb)
    ssem = (sem_sa, sem_sb)

    for phase in range(nph):
        sel = 2 * phase + c
        w = 32 * phase + c * NTILES + t

        zd = []
        for k in range(RPT // 16):
            off = pl.multiple_of(t * RPT + k * 16, 16)
            zd.append(pltpu.async_copy(zbuf, acc.at[pl.ds(off, 16)], sem_z))
        for d in zd:
            d.wait()
        plsc.subcore_barrier()

        for mw in range(NCHUNK // MWIN):
            moff = MWIN * mw
            pltpu.sync_copy(sp.at[w, pl.ds(moff, MWIN)], idx_s)
            pltpu.sync_copy(dp.at[w, pl.ds(moff, MWIN)], idx_d)
            for b in range(2):
                pltpu.async_copy(tbl.at[sel].at[idx_s.at[b]], bufs[b], gsem[b])

            def _pair(j, carry):
                for b in range(2):
                    k = 2 * j + b
                    pltpu.make_async_copy(tbl.at[sel].at[idx_s.at[k]],
                                          bufs[b], gsem[b]).wait()
                    sd = pltpu.async_copy(bufs[b], acc.at[idx_d.at[k]],
                                          ssem[b], add=True)
                    sd.wait()
                    kn = jnp.minimum(k + 2, MWIN - 1)
                    pltpu.async_copy(tbl.at[sel].at[idx_s.at[kn]],
                                     bufs[b], gsem[b])
                return carry
            lax.fori_loop(0, MWIN // 2, _pair, 0)
            for b in range(2):
                pltpu.make_async_copy(tbl.at[sel].at[idx_s.at[0]],
                                      bufs[b], gsem[b]).wait()
        plsc.subcore_barrier()

        pltpu.sync_copy(acc.at[pl.ds(tstart, RPT)],
                        out.at[sel, pl.ds(tstart, RPT)])


def _message_pass(tbl, sp, dp, nph):
    mesh = plsc.VectorSubcoreMesh(core_axis_name="c", subcore_axis_name="s")
    return pl.kernel(
        functools.partial(_msg_body, nph),
        out_type=jax.ShapeDtypeStruct((2 * nph, NU_PAD, 128), _f32),
        mesh=mesh,
        scratch_types=[
            pltpu.VMEM_SHARED((NU_PAD, 128), _f32),
            pltpu.VMEM((CHUNK, 128), _f32),
            pltpu.VMEM((CHUNK, 128), _f32),
            pltpu.VMEM((MWIN, CHUNK), _i32),
            pltpu.VMEM((MWIN, CHUNK), _i32),
            pltpu.VMEM((16, 128), _f32),
            pltpu.SemaphoreType.DMA,
            pltpu.SemaphoreType.DMA,
            pltpu.SemaphoreType.DMA,
            pltpu.SemaphoreType.DMA,
            pltpu.SemaphoreType.DMA,
        ],
    )(tbl, sp, dp)


_RB = 2048


def _tc1a_body(xu, xs, wu, ws, mu, ms):
    mu[...] = jnp.dot(xu[...], wu[...], preferred_element_type=_f32)
    ms[...] = jnp.dot(xs[...], ws[...], preferred_element_type=_f32)


def _tc1a(x_user, x_sub, w1u, w1s):
    return pl.pallas_call(
        _tc1a_body,
        grid=(NU_PAD // _RB,),
        in_specs=[
            pl.BlockSpec((_RB, D_IN), lambda i: (i, 0)),
            pl.BlockSpec((_RB, D_IN), lambda i: (i, 0)),
            pl.BlockSpec((D_IN, D_H), lambda i: (0, 0)),
            pl.BlockSpec((D_IN, D_H), lambda i: (0, 0)),
        ],
        out_specs=[
            pl.BlockSpec((_RB, D_H), lambda i: (i, 0)),
            pl.BlockSpec((_RB, D_H), lambda i: (i, 0)),
        ],
        out_shape=[
            jax.ShapeDtypeStruct((NU_PAD, D_H), _f32),
            jax.ShapeDtypeStruct((NU_PAD, D_H), _f32),
        ],
    )(x_user, x_sub, w1u, w1s)


def _tc1b_body(mu, ms, dg, tb, nrm):
    nb = lax.rsqrt(jnp.maximum(dg[...], 1.0))
    nrm[...] = nb
    hu = mu[...] * nb[:, 0:1]
    hs = ms[...] * nb[:, 2:3]
    tb[...] = jnp.stack([hu[:, 0:128], hu[:, 128:256],
                         hs[:, 0:128], hs[:, 128:256]])


def _tc1b(mu, ms, degT):
    return pl.pallas_call(
        _tc1b_body,
        grid=(NU_PAD // _RB,),
        in_specs=[
            pl.BlockSpec((_RB, D_H), lambda i: (i, 0)),
            pl.BlockSpec((_RB, D_H), lambda i: (i, 0)),
            pl.BlockSpec((_RB, 4), lambda i: (i, 0)),
        ],
        out_specs=[
            pl.BlockSpec((4, _RB, 128), lambda i: (0, i, 0)),
            pl.BlockSpec((_RB, 4), lambda i: (i, 0)),
        ],
        out_shape=[
            jax.ShapeDtypeStruct((4, TBL_N, 128), _f32),
            jax.ShapeDtypeStruct((NU_PAD, 4), _f32),
        ],
    )(mu, ms, degT)


def _tc2_body(m4, nrm, b1, wu, ws, tb2):
    nb = nrm[...]
    b = b1[...]
    h0 = jnp.maximum(m4[0] * nb[:, 1:2] + b[0:1, :], 0.0) * nb[:, 2:3]
    h1 = jnp.maximum(m4[1] * nb[:, 1:2] + b[1:2, :], 0.0) * nb[:, 2:3]
    h2 = jnp.maximum(m4[2] * nb[:, 3:4] + b[2:3, :], 0.0) * nb[:, 0:1]
    h3 = jnp.maximum(m4[3] * nb[:, 3:4] + b[3:4, :], 0.0) * nb[:, 0:1]
    gu = (jnp.dot(h2, wu[0:128, :], preferred_element_type=_f32)
          + jnp.dot(h3, wu[128:256, :], preferred_element_type=_f32))
    gs = (jnp.dot(h0, ws[0:128, :], preferred_element_type=_f32)
          + jnp.dot(h1, ws[128:256, :], preferred_element_type=_f32))
    tb2[...] = jnp.stack([gu, gs])


def _tc2(m4, nrmT, b1, w2u, w2s):
    return pl.pallas_call(
        _tc2_body,
        grid=(NU_PAD // _RB,),
        in_specs=[
            pl.BlockSpec((4, _RB, 128), lambda i: (0, i, 0)),
            pl.BlockSpec((_RB, 4), lambda i: (i, 0)),
            pl.BlockSpec((4, 128), lambda i: (0, 0)),
            pl.BlockSpec((D_H, D_OUT), lambda i: (0, 0)),
            pl.BlockSpec((D_H, D_OUT), lambda i: (0, 0)),
        ],
        out_specs=[
            pl.BlockSpec((2, _RB, 128), lambda i: (0, i, 0)),
        ],
        out_shape=[
            jax.ShapeDtypeStruct((2, TBL_N, 128), _f32),
        ],
    )(m4, nrmT, b1, w2u, w2s)


def _tc3_body(m2, nrm, b2, ou, osub):
    nb = nrm[...]
    osub[...] = m2[0] * nb[:, 1:2] + b2[0:1, :]
    ou[...] = m2[1] * nb[:, 3:4] + b2[1:2, :]


def _tc3(m2, nrmT, b2):
    return pl.pallas_call(
        _tc3_body,
        grid=(NU_PAD // _RB,),
        in_specs=[
            pl.BlockSpec((2, _RB, 128), lambda i: (0, i, 0)),
            pl.BlockSpec((_RB, 4), lambda i: (i, 0)),
            pl.BlockSpec((2, 128), lambda i: (0, 0)),
        ],
        out_specs=[
            pl.BlockSpec((_RB, 128), lambda i: (i, 0)),
            pl.BlockSpec((_RB, 128), lambda i: (i, 0)),
        ],
        out_shape=[
            jax.ShapeDtypeStruct((NU, 128), _f32),
            jax.ShapeDtypeStruct((NU, 128), _f32),
        ],
    )(m2, nrmT, b2)


def kernel(x_user, x_sub, edge_index_us, edge_index_su,
           W1_us, b1_us, W1_su, b1_su, W2_us, b2_us, W2_su, b2_su):
    sp_us = _pad_src(edge_index_us[0])
    dp_us = _pad_dst(edge_index_us[1])
    sp_su = _pad_src(edge_index_su[0])
    dp_su = _pad_dst(edge_index_su[1])

    sp12 = jnp.concatenate([sp_us, sp_su])
    dp12 = jnp.concatenate([dp_us, dp_su])
    mu, ms = _tc1a(x_user, x_sub, W1_us, W1_su)
    degs = _degrees(sp12, dp12).reshape(4, ACC_H)
    degT = degs[:, :NU].T

    tbl1, nrmT = _tc1b(mu, ms, degT)

    m4 = _message_pass(
        tbl1,
        jnp.concatenate([sp_us, sp_us, sp_su, sp_su]),
        jnp.concatenate([dp_us, dp_us, dp_su, dp_su]),
        nph=2)

    tbl2, = _tc2(m4, nrmT, jnp.concatenate([b1_us, b1_su]).reshape(4, 128),
                 W2_us, W2_su)

    m2 = _message_pass(tbl2, sp12, dp12, nph=1)

    out_user, out_sub = _tc3(m2, nrmT, jnp.stack([b2_us, b2_su]))
    return out_user, out_sub
